# Initial kernel scaffold; baseline (speedup 1.0000x reference)
import jax
import jax.numpy as jnp
from jax.experimental import pallas as pl


def kernel(x, edge_index, W1, b1, W2, b2):
    raise NotImplementedError("write your pallas kernel here")


# speedup vs baseline: 7.7998x; 7.7998x over previous
import functools

import jax
import jax.numpy as jnp
from jax import lax
from jax.experimental import pallas as pl
from jax.experimental.pallas import tpu as pltpu
from jax.experimental.pallas import tpu_sc as plsc

_NC = 2
_NS = 16
_NW = _NC * _NS
_G = 80


def _k1_body(src_hbm, dst_hbm, dpo_hbm, dpi_hbm, sv, dv, ho, hi, *, NP, EW):
    c = lax.axis_index("c")
    s = lax.axis_index("s")
    wid = c * _NS + s
    z16 = jnp.zeros((16,), jnp.float32)

    def zero_body(j, _):
        ho[pl.ds(j * 16, 16)] = z16
        hi[pl.ds(j * 16, 16)] = z16
        return 0

    lax.fori_loop(0, NP // 16, zero_body, 0)

    e0 = pl.multiple_of(wid * EW, 8)
    pltpu.sync_copy(src_hbm.at[pl.ds(e0, EW)], sv)
    pltpu.sync_copy(dst_hbm.at[pl.ds(e0, EW)], dv)
    ones = jnp.ones((16,), jnp.float32)

    def edge_body(j, _):
        si = sv[pl.ds(j * 16, 16)]
        di = dv[pl.ds(j * 16, 16)]
        plsc.addupdate_scatter(ho, [si], ones)
        plsc.addupdate_scatter(hi, [di], ones)
        return 0

    lax.fori_loop(0, EW // 16, edge_body, 0)
    pltpu.sync_copy(ho, dpo_hbm.at[wid])
    pltpu.sync_copy(hi, dpi_hbm.at[wid])


def _make_k1(NP, EW):
    mesh = plsc.VectorSubcoreMesh(core_axis_name="c", subcore_axis_name="s")
    return pl.kernel(
        functools.partial(_k1_body, NP=NP, EW=EW),
        out_type=(
            jax.ShapeDtypeStruct((_NW, NP), jnp.float32),
            jax.ShapeDtypeStruct((_NW, NP), jnp.float32),
        ),
        mesh=mesh,
        compiler_params=pltpu.CompilerParams(needs_layout_passes=False),
        scratch_types=[
            pltpu.VMEM((EW,), jnp.int32),
            pltpu.VMEM((EW,), jnp.int32),
            pltpu.VMEM((NP,), jnp.float32),
            pltpu.VMEM((NP,), jnp.float32),
        ],
    )


def _k2_body(dpo_ref, dpi_ref, x_ref, xn_ref, no_ref, ni_ref):
    deg_o = jnp.sum(dpo_ref[...], axis=0)
    deg_i = jnp.sum(dpi_ref[...], axis=0)
    no = lax.rsqrt(jnp.where(deg_o > 0.0, deg_o, 1.0))
    ni = lax.rsqrt(jnp.where(deg_i > 0.0, deg_i, 1.0))
    no_ref[...] = no[None, :]
    ni_ref[...] = ni[None, :]
    xn_ref[...] = x_ref[...] * no[:, None]


def _make_k2(NP, D):
    nb = NP // 128
    return pl.pallas_call(
        _k2_body,
        grid=(nb,),
        in_specs=[
            pl.BlockSpec((_NW, 128), lambda i: (0, i)),
            pl.BlockSpec((_NW, 128), lambda i: (0, i)),
            pl.BlockSpec((128, D), lambda i: (i, 0)),
        ],
        out_specs=[
            pl.BlockSpec((128, D), lambda i: (i, 0)),
            pl.BlockSpec((1, 128), lambda i: (0, i)),
            pl.BlockSpec((1, 128), lambda i: (0, i)),
        ],
        out_shape=[
            jax.ShapeDtypeStruct((NP, D), jnp.float32),
            jax.ShapeDtypeStruct((1, NP), jnp.float32),
            jax.ShapeDtypeStruct((1, NP), jnp.float32),
        ],
    )


def _k3_body(xn_hbm, src_hbm, dst_hbm, nin_hbm, zrows_hbm, agg_hbm, sp_hbm,
             sidxv, didxv, rows, ninv, sloc, agg_sh, sem, *, NP, EW, D):
    c = lax.axis_index("c")
    s = lax.axis_index("s")
    wid = c * _NS + s
    rpt = NP // _NS
    z16 = jnp.zeros((16,), jnp.float32)

    def zero_body(j, _):
        sloc[pl.ds(j * 16, 16)] = z16
        return 0

    lax.fori_loop(0, NP // 16, zero_body, 0)
    pltpu.sync_copy(nin_hbm, ninv)
    r0 = pl.multiple_of(s * rpt, 8)
    pltpu.sync_copy(zrows_hbm, agg_sh.at[pl.ds(r0, rpt)])
    plsc.subcore_barrier()

    def chunk(k, _):
        e0 = pl.multiple_of(wid * EW + k * _G, 8)
        pltpu.sync_copy(src_hbm.at[pl.ds(e0, _G)], sidxv)
        pltpu.sync_copy(dst_hbm.at[pl.ds(e0, _G)], didxv)
        cp = pltpu.async_copy(xn_hbm.at[sidxv], rows, sem)

        def sbody(j, _):
            si = sidxv[pl.ds(j * 16, 16)]
            di = didxv[pl.ds(j * 16, 16)]
            vals = plsc.load_gather(ninv, [di])
            plsc.addupdate_scatter(sloc, [si], vals)
            return 0

        lax.fori_loop(0, _G // 16, sbody, 0)
        cp.wait()
        pltpu.sync_copy(rows, agg_sh.at[didxv], add=True)
        return 0

    lax.fori_loop(0, EW // _G, chunk, 0)
    plsc.subcore_barrier()
    pltpu.sync_copy(agg_sh.at[pl.ds(r0, rpt)], agg_hbm.at[c, pl.ds(r0, rpt)])
    pltpu.sync_copy(sloc, sp_hbm.at[wid])


def _make_k3(NP, EW, D):
    mesh = plsc.VectorSubcoreMesh(core_axis_name="c", subcore_axis_name="s")
    return pl.kernel(
        functools.partial(_k3_body, NP=NP, EW=EW, D=D),
        out_type=(
            jax.ShapeDtypeStruct((_NC, NP, D), jnp.float32),
            jax.ShapeDtypeStruct((_NW, NP), jnp.float32),
        ),
        mesh=mesh,
        compiler_params=pltpu.CompilerParams(needs_layout_passes=False),
        scratch_types=[
            pltpu.VMEM((_G,), jnp.int32),
            pltpu.VMEM((_G,), jnp.int32),
            pltpu.VMEM((_G, D), jnp.float32),
            pltpu.VMEM((NP,), jnp.float32),
            pltpu.VMEM((NP,), jnp.float32),
            pltpu.VMEM_SHARED((NP, D), jnp.float32),
            pltpu.SemaphoreType.DMA,
        ],
    )


def _k4_body(aggA_ref, aggB_ref, sp_ref, no_ref, ni_ref, w1_ref, b1_ref,
             w2_ref, b2_ref, out_ref, vacc, *, nb, n_nodes):
    i = pl.program_id(0)

    @pl.when(i == 0)
    def _():
        vacc[...] = jnp.zeros_like(vacc)

    agg = aggA_ref[...] + aggB_ref[...]
    z = jnp.dot(agg, w1_ref[...], preferred_element_type=jnp.float32)
    h1 = jnp.maximum(z * ni_ref[0, :][:, None] + b1_ref[...], 0.0)
    w = no_ref[...] * jnp.sum(sp_ref[...], axis=0)[None, :]
    vacc[...] += jnp.dot(w, h1, preferred_element_type=jnp.float32)

    @pl.when(i == nb - 1)
    def _():
        v = vacc[...] * (1.0 / n_nodes)
        out_ref[...] = (
            jnp.dot(v, w2_ref[...], preferred_element_type=jnp.float32)
            + b2_ref[...]
        )


def _make_k4(NP, D, H, n_nodes):
    nb = NP // 128
    return pl.pallas_call(
        functools.partial(_k4_body, nb=nb, n_nodes=n_nodes),
        grid=(nb,),
        in_specs=[
            pl.BlockSpec((128, D), lambda i: (i, 0)),
            pl.BlockSpec((128, D), lambda i: (i, 0)),
            pl.BlockSpec((_NW, 128), lambda i: (0, i)),
            pl.BlockSpec((1, 128), lambda i: (0, i)),
            pl.BlockSpec((1, 128), lambda i: (0, i)),
            pl.BlockSpec((D, H), lambda i: (0, 0)),
            pl.BlockSpec((1, H), lambda i: (0, 0)),
            pl.BlockSpec((H, 128), lambda i: (0, 0)),
            pl.BlockSpec((1, 128), lambda i: (0, 0)),
        ],
        out_specs=pl.BlockSpec((1, 128), lambda i: (0, 0)),
        out_shape=jax.ShapeDtypeStruct((1, 128), jnp.float32),
        scratch_shapes=[pltpu.VMEM((1, 128), jnp.float32)],
    )


def kernel(x, edge_index, W1, b1, W2, b2):
    N, D = x.shape
    E = edge_index.shape[1]
    H = W1.shape[1]
    C = W2.shape[1]
    NP = -(-N // 2048) * 2048
    EW = E // _NW
    assert E % (_NW * _G) == 0 and NP % (_NS * 8) == 0

    xpad = jnp.pad(x, ((0, NP - N), (0, 0)))
    src = edge_index[0]
    dst = edge_index[1]
    dpo, dpi = _make_k1(NP, EW)(src, dst)
    xn, no_, ni_ = _make_k2(NP, D)(dpo, dpi, xpad)
    zrows = jnp.zeros((NP // _NS, D), jnp.float32)
    agg2, spart = _make_k3(NP, EW, D)(xn, src, dst, ni_.reshape(NP), zrows)
    w2p = jnp.pad(W2, ((0, 0), (0, 128 - C)))
    b2p = jnp.pad(b2, (0, 128 - C)).reshape(1, 128)
    out = _make_k4(NP, D, H, N)(
        agg2[0], agg2[1], spart, no_, ni_, W1, b1.reshape(1, H), w2p, b2p)
    return out[:, :C]


# speedup vs baseline: 13.8631x; 1.7774x over previous
import functools

import jax
import jax.numpy as jnp
from jax import lax
from jax.experimental import pallas as pl
from jax.experimental.pallas import tpu as pltpu
from jax.experimental.pallas import tpu_sc as plsc

_NC = 2
_NS = 16
_NW = _NC * _NS
_G = 96
_BR = 512


def _k1_body(src_hbm, dst_hbm, dpo_hbm, dpi_hbm, sv, dv, ho, hi, *, NP, EW):
    c = lax.axis_index("c")
    s = lax.axis_index("s")
    wid = c * _NS + s
    z16 = jnp.zeros((16,), jnp.float32)

    def zero_body(j, _):
        ho[pl.ds(j * 16, 16)] = z16
        hi[pl.ds(j * 16, 16)] = z16
        return 0

    lax.fori_loop(0, NP // 16, zero_body, 0)

    e0 = pl.multiple_of(wid * EW, 8)
    pltpu.sync_copy(src_hbm.at[pl.ds(e0, EW)], sv)
    pltpu.sync_copy(dst_hbm.at[pl.ds(e0, EW)], dv)
    ones = jnp.ones((16,), jnp.float32)

    def edge_body(j, _):
        si = sv[pl.ds(j * 16, 16)]
        di = dv[pl.ds(j * 16, 16)]
        plsc.addupdate_scatter(ho, [si], ones)
        plsc.addupdate_scatter(hi, [di], ones)
        return 0

    lax.fori_loop(0, EW // 16, edge_body, 0)
    pltpu.sync_copy(ho, dpo_hbm.at[wid])
    pltpu.sync_copy(hi, dpi_hbm.at[wid])


def _make_k1(NP, EW):
    mesh = plsc.VectorSubcoreMesh(core_axis_name="c", subcore_axis_name="s")
    return pl.kernel(
        functools.partial(_k1_body, NP=NP, EW=EW),
        out_type=(
            jax.ShapeDtypeStruct((_NW, NP), jnp.float32),
            jax.ShapeDtypeStruct((_NW, NP), jnp.float32),
        ),
        mesh=mesh,
        compiler_params=pltpu.CompilerParams(needs_layout_passes=False),
        scratch_types=[
            pltpu.VMEM((EW,), jnp.int32),
            pltpu.VMEM((EW,), jnp.int32),
            pltpu.VMEM((NP,), jnp.float32),
            pltpu.VMEM((NP,), jnp.float32),
        ],
    )


def _k2_body(dpo_ref, dpi_ref, x_ref, xn_ref, no_ref, ni_ref):
    deg_o = jnp.sum(dpo_ref[...], axis=0)
    deg_i = jnp.sum(dpi_ref[...], axis=0)
    no = lax.rsqrt(jnp.where(deg_o > 0.0, deg_o, 1.0))
    ni = lax.rsqrt(jnp.where(deg_i > 0.0, deg_i, 1.0))
    no_ref[...] = no[None, :]
    ni_ref[...] = ni[None, :]
    xn_ref[...] = x_ref[...] * no[:, None]


def _make_k2(NP, D):
    nb = NP // _BR
    return pl.pallas_call(
        _k2_body,
        grid=(nb,),
        in_specs=[
            pl.BlockSpec((_NW, _BR), lambda i: (0, i)),
            pl.BlockSpec((_NW, _BR), lambda i: (0, i)),
            pl.BlockSpec((_BR, D), lambda i: (i, 0)),
        ],
        out_specs=[
            pl.BlockSpec((_BR, D), lambda i: (i, 0)),
            pl.BlockSpec((1, _BR), lambda i: (0, i)),
            pl.BlockSpec((1, _BR), lambda i: (0, i)),
        ],
        out_shape=[
            jax.ShapeDtypeStruct((NP, D), jnp.float32),
            jax.ShapeDtypeStruct((1, NP), jnp.float32),
            jax.ShapeDtypeStruct((1, NP), jnp.float32),
        ],
    )


def _k3_body(xn_hbm, src_hbm, dst_hbm, nin_hbm, zrows_hbm, agg_hbm, sp_hbm,
             sidx0, didx0, rows0, sidx1, didx1, rows1, tsidx, tdidx,
             ninv, sloc, agg_sh, sem0, sem1, semt, *, NP, EW, D):
    c = lax.axis_index("c")
    s = lax.axis_index("s")
    wid = c * _NS + s
    rpt = NP // _NS
    z16 = jnp.zeros((16,), jnp.float32)
    ncf = (EW // _G) & ~1
    tail = EW - ncf * _G
    ebase = wid * EW

    def zero_body(j, _):
        sloc[pl.ds(j * 16, 16)] = z16
        return 0

    lax.fori_loop(0, NP // 16, zero_body, 0)
    pltpu.sync_copy(nin_hbm, ninv)
    r0 = pl.multiple_of(s * rpt, 8)
    pltpu.sync_copy(zrows_hbm, agg_sh.at[pl.ds(r0, rpt)])

    def load_idx(k, sv, dv):
        e0 = pl.multiple_of(ebase + k * _G, 8)
        pltpu.sync_copy(src_hbm.at[pl.ds(e0, _G)], sv)
        pltpu.sync_copy(dst_hbm.at[pl.ds(e0, _G)], dv)

    def spass(sv, dv, n):
        def sbody(j, _):
            si = sv[pl.ds(j * 16, 16)]
            di = dv[pl.ds(j * 16, 16)]
            vals = plsc.load_gather(ninv, [di])
            plsc.addupdate_scatter(sloc, [si], vals)
            return 0

        lax.fori_loop(0, n // 16, sbody, 0)

    load_idx(0, sidx0, didx0)
    pltpu.async_copy(xn_hbm.at[sidx0], rows0, sem0)
    plsc.subcore_barrier()

    def pair(p, _):
        k0 = p * 2
        load_idx(k0 + 1, sidx1, didx1)
        pltpu.async_copy(xn_hbm.at[sidx1], rows1, sem1)
        spass(sidx0, didx0, _G)
        pltpu.make_async_copy(xn_hbm.at[sidx0], rows0, sem0).wait()
        pltpu.sync_copy(rows0, agg_sh.at[didx0], add=True)

        @pl.when(k0 + 2 < ncf)
        def _():
            load_idx(k0 + 2, sidx0, didx0)
            pltpu.async_copy(xn_hbm.at[sidx0], rows0, sem0)

        spass(sidx1, didx1, _G)
        pltpu.make_async_copy(xn_hbm.at[sidx1], rows1, sem1).wait()
        pltpu.sync_copy(rows1, agg_sh.at[didx1], add=True)
        return 0

    lax.fori_loop(0, ncf // 2, pair, 0)

    if tail:
        e0 = pl.multiple_of(ebase + ncf * _G, 8)
        pltpu.sync_copy(src_hbm.at[pl.ds(e0, tail)], tsidx)
        pltpu.sync_copy(dst_hbm.at[pl.ds(e0, tail)], tdidx)
        cp = pltpu.async_copy(xn_hbm.at[tsidx], rows0.at[pl.ds(0, tail)], semt)
        spass(tsidx, tdidx, tail)
        cp.wait()
        pltpu.sync_copy(rows0.at[pl.ds(0, tail)], agg_sh.at[tdidx], add=True)

    plsc.subcore_barrier()
    pltpu.sync_copy(agg_sh.at[pl.ds(r0, rpt)], agg_hbm.at[c, pl.ds(r0, rpt)])
    pltpu.sync_copy(sloc, sp_hbm.at[wid])


def _make_k3(NP, EW, D):
    tail = EW - ((EW // _G) & ~1) * _G
    mesh = plsc.VectorSubcoreMesh(core_axis_name="c", subcore_axis_name="s")
    return pl.kernel(
        functools.partial(_k3_body, NP=NP, EW=EW, D=D),
        out_type=(
            jax.ShapeDtypeStruct((_NC, NP, D), jnp.float32),
            jax.ShapeDtypeStruct((_NW, NP), jnp.float32),
        ),
        mesh=mesh,
        compiler_params=pltpu.CompilerParams(needs_layout_passes=False),
        scratch_types=[
            pltpu.VMEM((_G,), jnp.int32),
            pltpu.VMEM((_G,), jnp.int32),
            pltpu.VMEM((_G, D), jnp.float32),
            pltpu.VMEM((_G,), jnp.int32),
            pltpu.VMEM((_G,), jnp.int32),
            pltpu.VMEM((_G, D), jnp.float32),
            pltpu.VMEM((max(tail, 8),), jnp.int32),
            pltpu.VMEM((max(tail, 8),), jnp.int32),
            pltpu.VMEM((NP,), jnp.float32),
            pltpu.VMEM((NP,), jnp.float32),
            pltpu.VMEM_SHARED((NP, D), jnp.float32),
            pltpu.SemaphoreType.DMA,
            pltpu.SemaphoreType.DMA,
            pltpu.SemaphoreType.DMA,
        ],
    )


def _k4_body(aggA_ref, aggB_ref, sp_ref, no_ref, ni_ref, w1_ref, b1_ref,
             w2_ref, b2_ref, out_ref, vacc, *, nb, n_nodes):
    i = pl.program_id(0)

    @pl.when(i == 0)
    def _():
        vacc[...] = jnp.zeros_like(vacc)

    agg = aggA_ref[...] + aggB_ref[...]
    z = jnp.dot(agg, w1_ref[...], preferred_element_type=jnp.float32)
    h1 = jnp.maximum(z * ni_ref[0, :][:, None] + b1_ref[...], 0.0)
    w = no_ref[...] * jnp.sum(sp_ref[...], axis=0)[None, :]
    vacc[...] += jnp.dot(w, h1, preferred_element_type=jnp.float32)

    @pl.when(i == nb - 1)
    def _():
        v = vacc[...] * (1.0 / n_nodes)
        out_ref[...] = (
            jnp.dot(v, w2_ref[...], preferred_element_type=jnp.float32)
            + b2_ref[...]
        )


def _make_k4(NP, D, H, n_nodes):
    nb = NP // _BR
    return pl.pallas_call(
        functools.partial(_k4_body, nb=nb, n_nodes=n_nodes),
        grid=(nb,),
        in_specs=[
            pl.BlockSpec((_BR, D), lambda i: (i, 0)),
            pl.BlockSpec((_BR, D), lambda i: (i, 0)),
            pl.BlockSpec((_NW, _BR), lambda i: (0, i)),
            pl.BlockSpec((1, _BR), lambda i: (0, i)),
            pl.BlockSpec((1, _BR), lambda i: (0, i)),
            pl.BlockSpec((D, H), lambda i: (0, 0)),
            pl.BlockSpec((1, H), lambda i: (0, 0)),
            pl.BlockSpec((H, 128), lambda i: (0, 0)),
            pl.BlockSpec((1, 128), lambda i: (0, 0)),
        ],
        out_specs=pl.BlockSpec((1, 128), lambda i: (0, 0)),
        out_shape=jax.ShapeDtypeStruct((1, 128), jnp.float32),
        scratch_shapes=[pltpu.VMEM((1, 128), jnp.float32)],
    )


def kernel(x, edge_index, W1, b1, W2, b2):
    N, D = x.shape
    E = edge_index.shape[1]
    H = W1.shape[1]
    C = W2.shape[1]
    NP = -(-N // 2048) * 2048
    EW = E // _NW
    assert E % (_NW * 8) == 0 and NP % (_NS * 8) == 0 and NP % _BR == 0

    xpad = jnp.pad(x, ((0, NP - N), (0, 0)))
    src = edge_index[0]
    dst = edge_index[1]
    dpo, dpi = _make_k1(NP, EW)(src, dst)
    xn, no_, ni_ = _make_k2(NP, D)(dpo, dpi, xpad)
    zrows = jnp.zeros((NP // _NS, D), jnp.float32)
    agg2, spart = _make_k3(NP, EW, D)(xn, src, dst, ni_.reshape(NP), zrows)
    w2p = jnp.pad(W2, ((0, 0), (0, 128 - C)))
    b2p = jnp.pad(b2, (0, 128 - C)).reshape(1, 128)
    out = _make_k4(NP, D, H, N)(
        agg2[0], agg2[1], spart, no_, ni_, W1, b1.reshape(1, H), w2p, b2p)
    return out[:, :C]


# speedup vs baseline: 15.9545x; 1.1509x over previous
import functools

import jax
import jax.numpy as jnp
from jax import lax
from jax.experimental import pallas as pl
from jax.experimental.pallas import tpu as pltpu
from jax.experimental.pallas import tpu_sc as plsc

_NC = 2
_NS = 16
_NW = _NC * _NS
_G = 96
_BR = 512


def _k1_body(src_hbm, dst_hbm, dpo_hbm, dpi_hbm, sv, dv, ho, hi, *, NP, EW):
    c = lax.axis_index("c")
    s = lax.axis_index("s")
    wid = c * _NS + s
    z16 = jnp.zeros((16,), jnp.float32)

    def zero_body(j, _):
        ho[pl.ds(j * 16, 16)] = z16
        hi[pl.ds(j * 16, 16)] = z16
        return 0

    lax.fori_loop(0, NP // 16, zero_body, 0)

    e0 = pl.multiple_of(wid * EW, 8)
    pltpu.sync_copy(src_hbm.at[pl.ds(e0, EW)], sv)
    pltpu.sync_copy(dst_hbm.at[pl.ds(e0, EW)], dv)
    ones = jnp.ones((16,), jnp.float32)

    def edge_body(j, _):
        si = sv[pl.ds(j * 16, 16)]
        di = dv[pl.ds(j * 16, 16)]
        plsc.addupdate_scatter(ho, [si], ones)
        plsc.addupdate_scatter(hi, [di], ones)
        return 0

    lax.fori_loop(0, EW // 16, edge_body, 0)
    pltpu.sync_copy(ho, dpo_hbm.at[wid])
    pltpu.sync_copy(hi, dpi_hbm.at[wid])


def _make_k1(NP, EW):
    mesh = plsc.VectorSubcoreMesh(core_axis_name="c", subcore_axis_name="s")
    return pl.kernel(
        functools.partial(_k1_body, NP=NP, EW=EW),
        out_type=(
            jax.ShapeDtypeStruct((_NW, NP), jnp.float32),
            jax.ShapeDtypeStruct((_NW, NP), jnp.float32),
        ),
        mesh=mesh,
        compiler_params=pltpu.CompilerParams(needs_layout_passes=False),
        scratch_types=[
            pltpu.VMEM((EW,), jnp.int32),
            pltpu.VMEM((EW,), jnp.int32),
            pltpu.VMEM((NP,), jnp.float32),
            pltpu.VMEM((NP,), jnp.float32),
        ],
    )


def _k2_body(dpo_ref, dpi_ref, x_ref, xn_ref, no_ref, ni_ref):
    deg_o = jnp.sum(dpo_ref[...], axis=0)
    deg_i = jnp.sum(dpi_ref[...], axis=0)
    no = lax.rsqrt(jnp.where(deg_o > 0.0, deg_o, 1.0))
    ni = lax.rsqrt(jnp.where(deg_i > 0.0, deg_i, 1.0))
    no_ref[...] = no[None, :]
    ni_ref[...] = ni[None, :]
    xn_ref[...] = x_ref[...] * no[:, None]


def _make_k2(NP, D):
    nb = NP // _BR
    return pl.pallas_call(
        _k2_body,
        grid=(nb,),
        in_specs=[
            pl.BlockSpec((_NW, _BR), lambda i: (0, i)),
            pl.BlockSpec((_NW, _BR), lambda i: (0, i)),
            pl.BlockSpec((_BR, D), lambda i: (i, 0)),
        ],
        out_specs=[
            pl.BlockSpec((_BR, D), lambda i: (i, 0)),
            pl.BlockSpec((1, _BR), lambda i: (0, i)),
            pl.BlockSpec((1, _BR), lambda i: (0, i)),
        ],
        out_shape=[
            jax.ShapeDtypeStruct((NP, D), jnp.float32),
            jax.ShapeDtypeStruct((1, NP), jnp.float32),
            jax.ShapeDtypeStruct((1, NP), jnp.float32),
        ],
    )


def _k3_body(xn_hbm, src_hbm, dst_hbm, nin_hbm, zrows_hbm, agg_hbm, sp_hbm,
             sidx0, didx0, rows0, sidx1, didx1, rows1, tsidx, tdidx,
             ninv, sloc, agg_sh, sem0, sem1, semt, semi0, semi1,
             *, NP, EW, D):
    c = lax.axis_index("c")
    s = lax.axis_index("s")
    wid = c * _NS + s
    rpt = NP // _NS
    z16 = jnp.zeros((16,), jnp.float32)
    ncf = (EW // _G) & ~1
    tail = EW - ncf * _G
    ebase = wid * EW

    def zero_body(j, _):
        sloc[pl.ds(j * 16, 16)] = z16
        return 0

    lax.fori_loop(0, NP // 16, zero_body, 0)
    pltpu.sync_copy(nin_hbm, ninv)
    r0 = pl.multiple_of(s * rpt, 8)
    pltpu.sync_copy(zrows_hbm, agg_sh.at[pl.ds(r0, rpt)])

    def fire_idx(k, sv, dv, semi):
        e0 = pl.multiple_of(ebase + k * _G, 8)
        pltpu.async_copy(src_hbm.at[pl.ds(e0, _G)], sv, semi)
        pltpu.async_copy(dst_hbm.at[pl.ds(e0, _G)], dv, semi)

    def wait_idx(sv, dv, semi):
        dummy = pl.multiple_of(ebase, 8)
        pltpu.make_async_copy(src_hbm.at[pl.ds(dummy, _G)], sv, semi).wait()
        pltpu.make_async_copy(dst_hbm.at[pl.ds(dummy, _G)], dv, semi).wait()

    def spass(sv, dv, n):
        def sbody(j, _):
            si = sv[pl.ds(j * 16, 16)]
            di = dv[pl.ds(j * 16, 16)]
            vals = plsc.load_gather(ninv, [di])
            plsc.addupdate_scatter(sloc, [si], vals)
            return 0

        lax.fori_loop(0, n // 16, sbody, 0)

    fire_idx(0, sidx0, didx0, semi0)
    wait_idx(sidx0, didx0, semi0)
    pltpu.async_copy(xn_hbm.at[sidx0], rows0, sem0)
    fire_idx(1, sidx1, didx1, semi1)
    plsc.subcore_barrier()

    def phase(k, cur, nxt):
        (csi, cdi, crows, csemi, csemg) = cur
        (nsi, ndi, nrows, nsemi, nsemg) = nxt

        @pl.when(k + 1 < ncf)
        def _():
            wait_idx(nsi, ndi, nsemi)
            pltpu.async_copy(xn_hbm.at[nsi], nrows, nsemg)

        spass(csi, cdi, _G)
        pltpu.make_async_copy(xn_hbm.at[csi], crows, csemg).wait()
        pltpu.sync_copy(crows, agg_sh.at[cdi], add=True)

        @pl.when(k + 2 < ncf)
        def _():
            fire_idx(k + 2, csi, cdi, csemi)

    bufA = (sidx0, didx0, rows0, semi0, sem0)
    bufB = (sidx1, didx1, rows1, semi1, sem1)

    def pair(p, _):
        phase(p * 2, bufA, bufB)
        phase(p * 2 + 1, bufB, bufA)
        return 0

    lax.fori_loop(0, ncf // 2, pair, 0)

    if tail:
        e0 = pl.multiple_of(ebase + ncf * _G, 8)
        pltpu.sync_copy(src_hbm.at[pl.ds(e0, tail)], tsidx)
        pltpu.sync_copy(dst_hbm.at[pl.ds(e0, tail)], tdidx)
        cp = pltpu.async_copy(xn_hbm.at[tsidx], rows0.at[pl.ds(0, tail)], semt)
        spass(tsidx, tdidx, tail)
        cp.wait()
        pltpu.sync_copy(rows0.at[pl.ds(0, tail)], agg_sh.at[tdidx], add=True)

    plsc.subcore_barrier()
    pltpu.sync_copy(agg_sh.at[pl.ds(r0, rpt)], agg_hbm.at[c, pl.ds(r0, rpt)])
    pltpu.sync_copy(sloc, sp_hbm.at[wid])


def _make_k3(NP, EW, D):
    tail = EW - ((EW // _G) & ~1) * _G
    mesh = plsc.VectorSubcoreMesh(core_axis_name="c", subcore_axis_name="s")
    return pl.kernel(
        functools.partial(_k3_body, NP=NP, EW=EW, D=D),
        out_type=(
            jax.ShapeDtypeStruct((_NC, NP, D), jnp.float32),
            jax.ShapeDtypeStruct((_NW, NP), jnp.float32),
        ),
        mesh=mesh,
        compiler_params=pltpu.CompilerParams(needs_layout_passes=False),
        scratch_types=[
            pltpu.VMEM((_G,), jnp.int32),
            pltpu.VMEM((_G,), jnp.int32),
            pltpu.VMEM((_G, D), jnp.float32),
            pltpu.VMEM((_G,), jnp.int32),
            pltpu.VMEM((_G,), jnp.int32),
            pltpu.VMEM((_G, D), jnp.float32),
            pltpu.VMEM((max(tail, 8),), jnp.int32),
            pltpu.VMEM((max(tail, 8),), jnp.int32),
            pltpu.VMEM((NP,), jnp.float32),
            pltpu.VMEM((NP,), jnp.float32),
            pltpu.VMEM_SHARED((NP, D), jnp.float32),
            pltpu.SemaphoreType.DMA,
            pltpu.SemaphoreType.DMA,
            pltpu.SemaphoreType.DMA,
            pltpu.SemaphoreType.DMA,
            pltpu.SemaphoreType.DMA,
        ],
    )


def _k4_body(aggA_ref, aggB_ref, sp_ref, no_ref, ni_ref, w1_ref, b1_ref,
             w2_ref, b2_ref, out_ref, vacc, *, nb, n_nodes):
    i = pl.program_id(0)

    @pl.when(i == 0)
    def _():
        vacc[...] = jnp.zeros_like(vacc)

    agg = aggA_ref[...] + aggB_ref[...]
    z = jnp.dot(agg, w1_ref[...], preferred_element_type=jnp.float32)
    h1 = jnp.maximum(z * ni_ref[0, :][:, None] + b1_ref[...], 0.0)
    w = no_ref[...] * jnp.sum(sp_ref[...], axis=0)[None, :]
    vacc[...] += jnp.dot(w, h1, preferred_element_type=jnp.float32)

    @pl.when(i == nb - 1)
    def _():
        v = vacc[...] * (1.0 / n_nodes)
        out_ref[...] = (
            jnp.dot(v, w2_ref[...], preferred_element_type=jnp.float32)
            + b2_ref[...]
        )


def _make_k4(NP, D, H, n_nodes):
    nb = NP // _BR
    return pl.pallas_call(
        functools.partial(_k4_body, nb=nb, n_nodes=n_nodes),
        grid=(nb,),
        in_specs=[
            pl.BlockSpec((_BR, D), lambda i: (i, 0)),
            pl.BlockSpec((_BR, D), lambda i: (i, 0)),
            pl.BlockSpec((_NW, _BR), lambda i: (0, i)),
            pl.BlockSpec((1, _BR), lambda i: (0, i)),
            pl.BlockSpec((1, _BR), lambda i: (0, i)),
            pl.BlockSpec((D, H), lambda i: (0, 0)),
            pl.BlockSpec((1, H), lambda i: (0, 0)),
            pl.BlockSpec((H, 128), lambda i: (0, 0)),
            pl.BlockSpec((1, 128), lambda i: (0, 0)),
        ],
        out_specs=pl.BlockSpec((1, 128), lambda i: (0, 0)),
        out_shape=jax.ShapeDtypeStruct((1, 128), jnp.float32),
        scratch_shapes=[pltpu.VMEM((1, 128), jnp.float32)],
    )


def kernel(x, edge_index, W1, b1, W2, b2):
    N, D = x.shape
    E = edge_index.shape[1]
    H = W1.shape[1]
    C = W2.shape[1]
    NP = -(-N // 2048) * 2048
    EW = E // _NW
    assert E % (_NW * 8) == 0 and NP % (_NS * 8) == 0 and NP % _BR == 0

    xpad = jnp.pad(x, ((0, NP - N), (0, 0)))
    src = edge_index[0]
    dst = edge_index[1]
    dpo, dpi = _make_k1(NP, EW)(src, dst)
    xn, no_, ni_ = _make_k2(NP, D)(dpo, dpi, xpad)
    zrows = jnp.zeros((NP // _NS, D), jnp.float32)
    agg2, spart = _make_k3(NP, EW, D)(xn, src, dst, ni_.reshape(NP), zrows)
    w2p = jnp.pad(W2, ((0, 0), (0, 128 - C)))
    b2p = jnp.pad(b2, (0, 128 - C)).reshape(1, 128)
    out = _make_k4(NP, D, H, N)(
        agg2[0], agg2[1], spart, no_, ni_, W1, b1.reshape(1, H), w2p, b2p)
    return out[:, :C]


# speedup vs baseline: 16.2959x; 1.0214x over previous
import functools

import jax
import jax.numpy as jnp
from jax import lax
from jax.experimental import pallas as pl
from jax.experimental.pallas import tpu as pltpu
from jax.experimental.pallas import tpu_sc as plsc

_NC = 2
_NS = 16
_NW = _NC * _NS
_G = 96
_BR = 512


def _k1_body(src_hbm, dst_hbm, dpo_hbm, dpi_hbm, sv, dv, ho, hi, sems, semd,
             *, NP, EW):
    c = lax.axis_index("c")
    s = lax.axis_index("s")
    wid = c * _NS + s
    z16 = jnp.zeros((16,), jnp.float32)

    e0 = pl.multiple_of(wid * EW, 8)
    cps = pltpu.async_copy(src_hbm.at[pl.ds(e0, EW)], sv, sems)
    cpd = pltpu.async_copy(dst_hbm.at[pl.ds(e0, EW)], dv, semd)

    def zero_body(j, _):
        for u in range(8):
            ho[pl.ds(j * 128 + u * 16, 16)] = z16
            hi[pl.ds(j * 128 + u * 16, 16)] = z16
        return 0

    lax.fori_loop(0, NP // 128, zero_body, 0)
    cps.wait()
    cpd.wait()
    ones = jnp.ones((16,), jnp.float32)

    def edge_body(j, _):
        for u in range(5):
            si = sv[pl.ds(j * 80 + u * 16, 16)]
            di = dv[pl.ds(j * 80 + u * 16, 16)]
            plsc.addupdate_scatter(ho, [si], ones)
            plsc.addupdate_scatter(hi, [di], ones)
        return 0

    lax.fori_loop(0, EW // 80, edge_body, 0)
    pltpu.sync_copy(ho, dpo_hbm.at[wid])
    pltpu.sync_copy(hi, dpi_hbm.at[wid])


def _make_k1(NP, EW):
    mesh = plsc.VectorSubcoreMesh(core_axis_name="c", subcore_axis_name="s")
    return pl.kernel(
        functools.partial(_k1_body, NP=NP, EW=EW),
        out_type=(
            jax.ShapeDtypeStruct((_NW, NP), jnp.float32),
            jax.ShapeDtypeStruct((_NW, NP), jnp.float32),
        ),
        mesh=mesh,
        compiler_params=pltpu.CompilerParams(needs_layout_passes=False),
        scratch_types=[
            pltpu.VMEM((EW,), jnp.int32),
            pltpu.VMEM((EW,), jnp.int32),
            pltpu.VMEM((NP,), jnp.float32),
            pltpu.VMEM((NP,), jnp.float32),
            pltpu.SemaphoreType.DMA,
            pltpu.SemaphoreType.DMA,
        ],
    )


def _k2_body(dpo_ref, dpi_ref, x_ref, xn_ref, no_ref, ni_ref):
    deg_o = jnp.sum(dpo_ref[...], axis=0)
    deg_i = jnp.sum(dpi_ref[...], axis=0)
    no = lax.rsqrt(jnp.where(deg_o > 0.0, deg_o, 1.0))
    ni = lax.rsqrt(jnp.where(deg_i > 0.0, deg_i, 1.0))
    no_ref[...] = no[None, :]
    ni_ref[...] = ni[None, :]
    xn_ref[...] = x_ref[...] * no[:, None]


def _make_k2(NP, D):
    nb = NP // _BR
    return pl.pallas_call(
        _k2_body,
        grid=(nb,),
        in_specs=[
            pl.BlockSpec((_NW, _BR), lambda i: (0, i)),
            pl.BlockSpec((_NW, _BR), lambda i: (0, i)),
            pl.BlockSpec((_BR, D), lambda i: (i, 0)),
        ],
        out_specs=[
            pl.BlockSpec((_BR, D), lambda i: (i, 0)),
            pl.BlockSpec((1, _BR), lambda i: (0, i)),
            pl.BlockSpec((1, _BR), lambda i: (0, i)),
        ],
        out_shape=[
            jax.ShapeDtypeStruct((NP, D), jnp.float32),
            jax.ShapeDtypeStruct((1, NP), jnp.float32),
            jax.ShapeDtypeStruct((1, NP), jnp.float32),
        ],
    )


def _k3_body(xn_hbm, src_hbm, dst_hbm, nin_hbm, zrows_hbm, agg_hbm, sp_hbm,
             sidx0, didx0, rows0, sidx1, didx1, rows1, tsidx, tdidx,
             ninv, sloc, agg_sh, sem0, sem1, semt, semi0, semi1,
             *, NP, EW, D):
    c = lax.axis_index("c")
    s = lax.axis_index("s")
    wid = c * _NS + s
    rpt = NP // _NS
    z16 = jnp.zeros((16,), jnp.float32)
    ncf = (EW // _G) & ~1
    tail = EW - ncf * _G
    ebase = wid * EW

    def zero_body(j, _):
        for u in range(8):
            sloc[pl.ds(j * 128 + u * 16, 16)] = z16
        return 0

    lax.fori_loop(0, NP // 128, zero_body, 0)
    pltpu.sync_copy(nin_hbm, ninv)
    r0 = pl.multiple_of(s * rpt, 8)
    pltpu.sync_copy(zrows_hbm, agg_sh.at[pl.ds(r0, rpt)])

    def fire_idx(k, sv, dv, semi):
        e0 = pl.multiple_of(ebase + k * _G, 8)
        pltpu.async_copy(src_hbm.at[pl.ds(e0, _G)], sv, semi)
        pltpu.async_copy(dst_hbm.at[pl.ds(e0, _G)], dv, semi)

    def wait_idx(sv, dv, semi):
        dummy = pl.multiple_of(ebase, 8)
        pltpu.make_async_copy(src_hbm.at[pl.ds(dummy, _G)], sv, semi).wait()
        pltpu.make_async_copy(dst_hbm.at[pl.ds(dummy, _G)], dv, semi).wait()

    def spass(sv, dv, n):
        for j in range(n // 16):
            si = sv[pl.ds(j * 16, 16)]
            di = dv[pl.ds(j * 16, 16)]
            vals = plsc.load_gather(ninv, [di])
            plsc.addupdate_scatter(sloc, [si], vals)

    fire_idx(0, sidx0, didx0, semi0)
    wait_idx(sidx0, didx0, semi0)
    pltpu.async_copy(xn_hbm.at[sidx0], rows0, sem0)
    fire_idx(1, sidx1, didx1, semi1)
    plsc.subcore_barrier()

    def phase(k, cur, nxt):
        (csi, cdi, crows, csemi, csemg) = cur
        (nsi, ndi, nrows, nsemi, nsemg) = nxt

        @pl.when(k + 1 < ncf)
        def _():
            wait_idx(nsi, ndi, nsemi)
            pltpu.async_copy(xn_hbm.at[nsi], nrows, nsemg)

        spass(csi, cdi, _G)
        pltpu.make_async_copy(xn_hbm.at[csi], crows, csemg).wait()
        pltpu.sync_copy(crows, agg_sh.at[cdi], add=True)

        @pl.when(k + 2 < ncf)
        def _():
            fire_idx(k + 2, csi, cdi, csemi)

    bufA = (sidx0, didx0, rows0, semi0, sem0)
    bufB = (sidx1, didx1, rows1, semi1, sem1)

    def pair(p, _):
        phase(p * 2, bufA, bufB)
        phase(p * 2 + 1, bufB, bufA)
        return 0

    lax.fori_loop(0, ncf // 2, pair, 0)

    if tail:
        e0 = pl.multiple_of(ebase + ncf * _G, 8)
        pltpu.sync_copy(src_hbm.at[pl.ds(e0, tail)], tsidx)
        pltpu.sync_copy(dst_hbm.at[pl.ds(e0, tail)], tdidx)
        cp = pltpu.async_copy(xn_hbm.at[tsidx], rows0.at[pl.ds(0, tail)], semt)
        spass(tsidx, tdidx, tail)
        cp.wait()
        pltpu.sync_copy(rows0.at[pl.ds(0, tail)], agg_sh.at[tdidx], add=True)

    plsc.subcore_barrier()
    pltpu.sync_copy(agg_sh.at[pl.ds(r0, rpt)], agg_hbm.at[c, pl.ds(r0, rpt)])
    pltpu.sync_copy(sloc, sp_hbm.at[wid])


def _make_k3(NP, EW, D):
    tail = EW - ((EW // _G) & ~1) * _G
    mesh = plsc.VectorSubcoreMesh(core_axis_name="c", subcore_axis_name="s")
    return pl.kernel(
        functools.partial(_k3_body, NP=NP, EW=EW, D=D),
        out_type=(
            jax.ShapeDtypeStruct((_NC, NP, D), jnp.float32),
            jax.ShapeDtypeStruct((_NW, NP), jnp.float32),
        ),
        mesh=mesh,
        compiler_params=pltpu.CompilerParams(needs_layout_passes=False),
        scratch_types=[
            pltpu.VMEM((_G,), jnp.int32),
            pltpu.VMEM((_G,), jnp.int32),
            pltpu.VMEM((_G, D), jnp.float32),
            pltpu.VMEM((_G,), jnp.int32),
            pltpu.VMEM((_G,), jnp.int32),
            pltpu.VMEM((_G, D), jnp.float32),
            pltpu.VMEM((max(tail, 8),), jnp.int32),
            pltpu.VMEM((max(tail, 8),), jnp.int32),
            pltpu.VMEM((NP,), jnp.float32),
            pltpu.VMEM((NP,), jnp.float32),
            pltpu.VMEM_SHARED((NP, D), jnp.float32),
            pltpu.SemaphoreType.DMA,
            pltpu.SemaphoreType.DMA,
            pltpu.SemaphoreType.DMA,
            pltpu.SemaphoreType.DMA,
            pltpu.SemaphoreType.DMA,
        ],
    )


def _k4_body(aggA_ref, aggB_ref, sp_ref, no_ref, ni_ref, w1_ref, b1_ref,
             w2_ref, b2_ref, out_ref, vacc, *, nb, n_nodes):
    i = pl.program_id(0)

    @pl.when(i == 0)
    def _():
        vacc[...] = jnp.zeros_like(vacc)

    agg = aggA_ref[...] + aggB_ref[...]
    z = jnp.dot(agg, w1_ref[...], preferred_element_type=jnp.float32)
    h1 = jnp.maximum(z * ni_ref[0, :][:, None] + b1_ref[...], 0.0)
    w = no_ref[...] * jnp.sum(sp_ref[...], axis=0)[None, :]
    vacc[...] += jnp.dot(w, h1, preferred_element_type=jnp.float32)

    @pl.when(i == nb - 1)
    def _():
        v = vacc[...] * (1.0 / n_nodes)
        out_ref[...] = (
            jnp.dot(v, w2_ref[...], preferred_element_type=jnp.float32)
            + b2_ref[...]
        )


def _make_k4(NP, D, H, n_nodes):
    nb = NP // _BR
    return pl.pallas_call(
        functools.partial(_k4_body, nb=nb, n_nodes=n_nodes),
        grid=(nb,),
        in_specs=[
            pl.BlockSpec((_BR, D), lambda i: (i, 0)),
            pl.BlockSpec((_BR, D), lambda i: (i, 0)),
            pl.BlockSpec((_NW, _BR), lambda i: (0, i)),
            pl.BlockSpec((1, _BR), lambda i: (0, i)),
            pl.BlockSpec((1, _BR), lambda i: (0, i)),
            pl.BlockSpec((D, H), lambda i: (0, 0)),
            pl.BlockSpec((1, H), lambda i: (0, 0)),
            pl.BlockSpec((H, 128), lambda i: (0, 0)),
            pl.BlockSpec((1, 128), lambda i: (0, 0)),
        ],
        out_specs=pl.BlockSpec((1, 128), lambda i: (0, 0)),
        out_shape=jax.ShapeDtypeStruct((1, 128), jnp.float32),
        scratch_shapes=[pltpu.VMEM((1, 128), jnp.float32)],
    )


def kernel(x, edge_index, W1, b1, W2, b2):
    N, D = x.shape
    E = edge_index.shape[1]
    H = W1.shape[1]
    C = W2.shape[1]
    NP = -(-N // 2048) * 2048
    EW = E // _NW
    assert E % (_NW * 8) == 0 and NP % (_NS * 8) == 0 and NP % _BR == 0

    xpad = jnp.pad(x, ((0, NP - N), (0, 0)))
    src = edge_index[0]
    dst = edge_index[1]
    dpo, dpi = _make_k1(NP, EW)(src, dst)
    xn, no_, ni_ = _make_k2(NP, D)(dpo, dpi, xpad)
    zrows = jnp.zeros((NP // _NS, D), jnp.float32)
    agg2, spart = _make_k3(NP, EW, D)(xn, src, dst, ni_.reshape(NP), zrows)
    w2p = jnp.pad(W2, ((0, 0), (0, 128 - C)))
    b2p = jnp.pad(b2, (0, 128 - C)).reshape(1, 128)
    out = _make_k4(NP, D, H, N)(
        agg2[0], agg2[1], spart, no_, ni_, W1, b1.reshape(1, H), w2p, b2p)
    return out[:, :C]


# speedup vs baseline: 17.9558x; 1.1019x over previous
import functools

import jax
import jax.numpy as jnp
from jax import lax
from jax.experimental import pallas as pl
from jax.experimental.pallas import tpu as pltpu
from jax.experimental.pallas import tpu_sc as plsc

_NC = 2
_NS = 16
_NW = _NC * _NS
_G = 96
_BR = 512


def _k1_body(src_hbm, dst_hbm, dpo_hbm, dpi_hbm, sv, dv, ho, hi, sems, semd,
             *, NP, EW):
    c = lax.axis_index("c")
    s = lax.axis_index("s")
    wid = c * _NS + s
    z16 = jnp.zeros((16,), jnp.float32)

    e0 = pl.multiple_of(wid * EW, 8)
    cps = pltpu.async_copy(src_hbm.at[pl.ds(e0, EW)], sv, sems)
    cpd = pltpu.async_copy(dst_hbm.at[pl.ds(e0, EW)], dv, semd)

    def zero_body(j, _):
        for u in range(8):
            ho[pl.ds(j * 128 + u * 16, 16)] = z16
            hi[pl.ds(j * 128 + u * 16, 16)] = z16
        return 0

    lax.fori_loop(0, NP // 128, zero_body, 0)
    cps.wait()
    cpd.wait()
    ones = jnp.ones((16,), jnp.float32)

    def edge_body(j, _):
        for u in range(5):
            si = sv[pl.ds(j * 80 + u * 16, 16)]
            di = dv[pl.ds(j * 80 + u * 16, 16)]
            plsc.addupdate_scatter(ho, [si], ones)
            plsc.addupdate_scatter(hi, [di], ones)
        return 0

    lax.fori_loop(0, EW // 80, edge_body, 0)
    pltpu.sync_copy(ho, dpo_hbm.at[wid])
    pltpu.sync_copy(hi, dpi_hbm.at[wid])


def _make_k1(NP, EW):
    mesh = plsc.VectorSubcoreMesh(core_axis_name="c", subcore_axis_name="s")
    return pl.kernel(
        functools.partial(_k1_body, NP=NP, EW=EW),
        out_type=(
            jax.ShapeDtypeStruct((_NW, NP), jnp.float32),
            jax.ShapeDtypeStruct((_NW, NP), jnp.float32),
        ),
        mesh=mesh,
        compiler_params=pltpu.CompilerParams(needs_layout_passes=False),
        scratch_types=[
            pltpu.VMEM((EW,), jnp.int32),
            pltpu.VMEM((EW,), jnp.int32),
            pltpu.VMEM((NP,), jnp.float32),
            pltpu.VMEM((NP,), jnp.float32),
            pltpu.SemaphoreType.DMA,
            pltpu.SemaphoreType.DMA,
        ],
    )


def _k2_body(dpo_ref, dpi_ref, x_ref, xn_ref, no_ref, ni_ref):
    deg_o = jnp.sum(dpo_ref[...], axis=0)
    deg_i = jnp.sum(dpi_ref[...], axis=0)
    no = lax.rsqrt(jnp.where(deg_o > 0.0, deg_o, 1.0))
    ni = lax.rsqrt(jnp.where(deg_i > 0.0, deg_i, 1.0))
    no_ref[...] = no[None, :]
    ni_ref[...] = ni[None, :]
    xn_ref[...] = x_ref[...] * no[:, None]


def _make_k2(NP, D):
    nb = NP // _BR
    return pl.pallas_call(
        _k2_body,
        grid=(nb,),
        in_specs=[
            pl.BlockSpec((_NW, _BR), lambda i: (0, i)),
            pl.BlockSpec((_NW, _BR), lambda i: (0, i)),
            pl.BlockSpec((_BR, D), lambda i: (i, 0)),
        ],
        out_specs=[
            pl.BlockSpec((_BR, D), lambda i: (i, 0)),
            pl.BlockSpec((1, _BR), lambda i: (0, i)),
            pl.BlockSpec((1, _BR), lambda i: (0, i)),
        ],
        out_shape=[
            jax.ShapeDtypeStruct((NP, D), jnp.float32),
            jax.ShapeDtypeStruct((1, NP), jnp.float32),
            jax.ShapeDtypeStruct((1, NP), jnp.float32),
        ],
    )


def _k3_body(xn_hbm, src_hbm, dst_hbm, nin_hbm, zrows_hbm, agg_hbm, sp_hbm,
             sidx0, didx0, rows0, sidx1, didx1, rows1, tsidx, tdidx,
             didxs0, didxs1, ninv, sloc, agg_sh, sem0, sem1, semt,
             semi0, semi1, semsc0, semsc1, *, NP, EW, D):
    c = lax.axis_index("c")
    s = lax.axis_index("s")
    wid = c * _NS + s
    rpt = NP // _NS
    z16 = jnp.zeros((16,), jnp.float32)
    ncf = (EW // _G) & ~1
    tail = EW - ncf * _G
    ebase = wid * EW

    def zero_body(j, _):
        for u in range(8):
            sloc[pl.ds(j * 128 + u * 16, 16)] = z16
        return 0

    lax.fori_loop(0, NP // 128, zero_body, 0)
    pltpu.sync_copy(nin_hbm, ninv)
    r0 = pl.multiple_of(s * rpt, 8)
    pltpu.sync_copy(zrows_hbm, agg_sh.at[pl.ds(r0, rpt)])

    def fire_idx(k, sv, dv, semi):
        e0 = pl.multiple_of(ebase + k * _G, 8)
        pltpu.async_copy(src_hbm.at[pl.ds(e0, _G)], sv, semi)
        pltpu.async_copy(dst_hbm.at[pl.ds(e0, _G)], dv, semi)

    def wait_idx(sv, dv, semi):
        dummy = pl.multiple_of(ebase, 8)
        pltpu.make_async_copy(src_hbm.at[pl.ds(dummy, _G)], sv, semi).wait()
        pltpu.make_async_copy(dst_hbm.at[pl.ds(dummy, _G)], dv, semi).wait()

    def spass(sv, dv, n):
        for j in range(n // 16):
            si = sv[pl.ds(j * 16, 16)]
            di = dv[pl.ds(j * 16, 16)]
            vals = plsc.load_gather(ninv, [di])
            plsc.addupdate_scatter(sloc, [si], vals)

    fire_idx(0, sidx0, didx0, semi0)
    wait_idx(sidx0, didx0, semi0)
    pltpu.async_copy(xn_hbm.at[sidx0], rows0, sem0)
    fire_idx(1, sidx1, didx1, semi1)
    plsc.subcore_barrier()

    def phase(k, cur, nxt):
        (csi, cdi, crows, cdis, csemi, csemg, csemsc) = cur
        (nsi, ndi, nrows, ndis, nsemi, nsemg, nsemsc) = nxt

        @pl.when(k + 1 < ncf)
        def _():
            @pl.when(k >= 1)
            def _():
                pltpu.make_async_copy(nrows, agg_sh.at[ndis], nsemsc).wait()

            wait_idx(nsi, ndi, nsemi)
            pltpu.async_copy(xn_hbm.at[nsi], nrows, nsemg)

        spass(csi, cdi, _G)
        pltpu.make_async_copy(xn_hbm.at[csi], crows, csemg).wait()
        for j in range(_G // 16):
            cdis[pl.ds(j * 16, 16)] = cdi[pl.ds(j * 16, 16)]
        pltpu.async_copy(crows, agg_sh.at[cdis], csemsc, add=True)

        @pl.when(k + 2 < ncf)
        def _():
            fire_idx(k + 2, csi, cdi, csemi)

    bufA = (sidx0, didx0, rows0, didxs0, semi0, sem0, semsc0)
    bufB = (sidx1, didx1, rows1, didxs1, semi1, sem1, semsc1)

    def pair(p, _):
        phase(p * 2, bufA, bufB)
        phase(p * 2 + 1, bufB, bufA)
        return 0

    lax.fori_loop(0, ncf // 2, pair, 0)
    pltpu.make_async_copy(rows0, agg_sh.at[didxs0], semsc0).wait()
    pltpu.make_async_copy(rows1, agg_sh.at[didxs1], semsc1).wait()

    if tail:
        e0 = pl.multiple_of(ebase + ncf * _G, 8)
        pltpu.sync_copy(src_hbm.at[pl.ds(e0, tail)], tsidx)
        pltpu.sync_copy(dst_hbm.at[pl.ds(e0, tail)], tdidx)
        cp = pltpu.async_copy(xn_hbm.at[tsidx], rows0.at[pl.ds(0, tail)], semt)
        spass(tsidx, tdidx, tail)
        cp.wait()
        pltpu.sync_copy(rows0.at[pl.ds(0, tail)], agg_sh.at[tdidx], add=True)

    plsc.subcore_barrier()
    pltpu.sync_copy(agg_sh.at[pl.ds(r0, rpt)], agg_hbm.at[c, pl.ds(r0, rpt)])
    pltpu.sync_copy(sloc, sp_hbm.at[wid])


def _make_k3(NP, EW, D):
    tail = EW - ((EW // _G) & ~1) * _G
    mesh = plsc.VectorSubcoreMesh(core_axis_name="c", subcore_axis_name="s")
    return pl.kernel(
        functools.partial(_k3_body, NP=NP, EW=EW, D=D),
        out_type=(
            jax.ShapeDtypeStruct((_NC, NP, D), jnp.float32),
            jax.ShapeDtypeStruct((_NW, NP), jnp.float32),
        ),
        mesh=mesh,
        compiler_params=pltpu.CompilerParams(needs_layout_passes=False),
        scratch_types=[
            pltpu.VMEM((_G,), jnp.int32),
            pltpu.VMEM((_G,), jnp.int32),
            pltpu.VMEM((_G, D), jnp.float32),
            pltpu.VMEM((_G,), jnp.int32),
            pltpu.VMEM((_G,), jnp.int32),
            pltpu.VMEM((_G, D), jnp.float32),
            pltpu.VMEM((max(tail, 8),), jnp.int32),
            pltpu.VMEM((max(tail, 8),), jnp.int32),
            pltpu.VMEM((_G,), jnp.int32),
            pltpu.VMEM((_G,), jnp.int32),
            pltpu.VMEM((NP,), jnp.float32),
            pltpu.VMEM((NP,), jnp.float32),
            pltpu.VMEM_SHARED((NP, D), jnp.float32),
            pltpu.SemaphoreType.DMA,
            pltpu.SemaphoreType.DMA,
            pltpu.SemaphoreType.DMA,
            pltpu.SemaphoreType.DMA,
            pltpu.SemaphoreType.DMA,
            pltpu.SemaphoreType.DMA,
            pltpu.SemaphoreType.DMA,
        ],
    )


def _k4_body(aggA_ref, aggB_ref, sp_ref, no_ref, ni_ref, w1_ref, b1_ref,
             w2_ref, b2_ref, out_ref, vacc, *, nb, n_nodes):
    i = pl.program_id(0)

    @pl.when(i == 0)
    def _():
        vacc[...] = jnp.zeros_like(vacc)

    agg = aggA_ref[...] + aggB_ref[...]
    z = jnp.dot(agg, w1_ref[...], preferred_element_type=jnp.float32)
    h1 = jnp.maximum(z * ni_ref[0, :][:, None] + b1_ref[...], 0.0)
    w = no_ref[...] * jnp.sum(sp_ref[...], axis=0)[None, :]
    vacc[...] += jnp.dot(w, h1, preferred_element_type=jnp.float32)

    @pl.when(i == nb - 1)
    def _():
        v = vacc[...] * (1.0 / n_nodes)
        out_ref[...] = (
            jnp.dot(v, w2_ref[...], preferred_element_type=jnp.float32)
            + b2_ref[...]
        )


def _make_k4(NP, D, H, n_nodes):
    nb = NP // _BR
    return pl.pallas_call(
        functools.partial(_k4_body, nb=nb, n_nodes=n_nodes),
        grid=(nb,),
        in_specs=[
            pl.BlockSpec((_BR, D), lambda i: (i, 0)),
            pl.BlockSpec((_BR, D), lambda i: (i, 0)),
            pl.BlockSpec((_NW, _BR), lambda i: (0, i)),
            pl.BlockSpec((1, _BR), lambda i: (0, i)),
            pl.BlockSpec((1, _BR), lambda i: (0, i)),
            pl.BlockSpec((D, H), lambda i: (0, 0)),
            pl.BlockSpec((1, H), lambda i: (0, 0)),
            pl.BlockSpec((H, 128), lambda i: (0, 0)),
            pl.BlockSpec((1, 128), lambda i: (0, 0)),
        ],
        out_specs=pl.BlockSpec((1, 128), lambda i: (0, 0)),
        out_shape=jax.ShapeDtypeStruct((1, 128), jnp.float32),
        scratch_shapes=[pltpu.VMEM((1, 128), jnp.float32)],
    )


def kernel(x, edge_index, W1, b1, W2, b2):
    N, D = x.shape
    E = edge_index.shape[1]
    H = W1.shape[1]
    C = W2.shape[1]
    NP = -(-N // 2048) * 2048
    EW = E // _NW
    assert E % (_NW * 8) == 0 and NP % (_NS * 8) == 0 and NP % _BR == 0

    xpad = jnp.pad(x, ((0, NP - N), (0, 0)))
    src = edge_index[0]
    dst = edge_index[1]
    dpo, dpi = _make_k1(NP, EW)(src, dst)
    xn, no_, ni_ = _make_k2(NP, D)(dpo, dpi, xpad)
    zrows = jnp.zeros((NP // _NS, D), jnp.float32)
    agg2, spart = _make_k3(NP, EW, D)(xn, src, dst, ni_.reshape(NP), zrows)
    w2p = jnp.pad(W2, ((0, 0), (0, 128 - C)))
    b2p = jnp.pad(b2, (0, 128 - C)).reshape(1, 128)
    out = _make_k4(NP, D, H, N)(
        agg2[0], agg2[1], spart, no_, ni_, W1, b1.reshape(1, H), w2p, b2p)
    return out[:, :C]


# speedup vs baseline: 20.6670x; 1.1510x over previous
import functools

import jax
import jax.numpy as jnp
from jax import lax
from jax.experimental import pallas as pl
from jax.experimental.pallas import tpu as pltpu
from jax.experimental.pallas import tpu_sc as plsc

_NC = 2
_NS = 16
_NW = _NC * _NS
_G = 96
_BR = 1024


def _k1_body(src_hbm, dst_hbm, dpo_hbm, dpi_hbm, sv, dv, ho, hi, sems, semd,
             *, NP, EW):
    c = lax.axis_index("c")
    s = lax.axis_index("s")
    wid = c * _NS + s
    z16 = jnp.zeros((16,), jnp.float32)

    e0 = pl.multiple_of(wid * EW, 8)
    cps = pltpu.async_copy(src_hbm.at[pl.ds(e0, EW)], sv, sems)
    cpd = pltpu.async_copy(dst_hbm.at[pl.ds(e0, EW)], dv, semd)

    def zero_body(j, _):
        for u in range(8):
            ho[pl.ds(j * 128 + u * 16, 16)] = z16
            hi[pl.ds(j * 128 + u * 16, 16)] = z16
        return 0

    lax.fori_loop(0, NP // 128, zero_body, 0)
    cps.wait()
    cpd.wait()
    ones = jnp.ones((16,), jnp.float32)

    def edge_body(j, _):
        for u in range(5):
            si = sv[pl.ds(j * 80 + u * 16, 16)]
            di = dv[pl.ds(j * 80 + u * 16, 16)]
            plsc.addupdate_scatter(ho, [si], ones)
            plsc.addupdate_scatter(hi, [di], ones)
        return 0

    lax.fori_loop(0, EW // 80, edge_body, 0)
    pltpu.sync_copy(ho, dpo_hbm.at[wid])
    pltpu.sync_copy(hi, dpi_hbm.at[wid])


def _make_k1(NP, EW):
    mesh = plsc.VectorSubcoreMesh(core_axis_name="c", subcore_axis_name="s")
    return pl.kernel(
        functools.partial(_k1_body, NP=NP, EW=EW),
        out_type=(
            jax.ShapeDtypeStruct((_NW, NP), jnp.float32),
            jax.ShapeDtypeStruct((_NW, NP), jnp.float32),
        ),
        mesh=mesh,
        compiler_params=pltpu.CompilerParams(needs_layout_passes=False),
        scratch_types=[
            pltpu.VMEM((EW,), jnp.int32),
            pltpu.VMEM((EW,), jnp.int32),
            pltpu.VMEM((NP,), jnp.float32),
            pltpu.VMEM((NP,), jnp.float32),
            pltpu.SemaphoreType.DMA,
            pltpu.SemaphoreType.DMA,
        ],
    )


def _k2_body(dpo_ref, dpi_ref, x_ref, xn_ref, no_ref, ni_ref):
    deg_o = jnp.sum(dpo_ref[...], axis=0)
    deg_i = jnp.sum(dpi_ref[...], axis=0)
    no = lax.rsqrt(jnp.where(deg_o > 0.0, deg_o, 1.0))
    ni = lax.rsqrt(jnp.where(deg_i > 0.0, deg_i, 1.0))
    no_ref[...] = no[None, :]
    ni_ref[...] = ni[None, :]
    xn_ref[...] = x_ref[...] * no[:, None]


def _make_k2(NP, D):
    nb = NP // _BR
    return pl.pallas_call(
        _k2_body,
        grid=(nb,),
        in_specs=[
            pl.BlockSpec((_NW, _BR), lambda i: (0, i)),
            pl.BlockSpec((_NW, _BR), lambda i: (0, i)),
            pl.BlockSpec((_BR, D), lambda i: (i, 0)),
        ],
        out_specs=[
            pl.BlockSpec((_BR, D), lambda i: (i, 0)),
            pl.BlockSpec((1, _BR), lambda i: (0, i)),
            pl.BlockSpec((1, _BR), lambda i: (0, i)),
        ],
        out_shape=[
            jax.ShapeDtypeStruct((NP, D), jnp.float32),
            jax.ShapeDtypeStruct((1, NP), jnp.float32),
            jax.ShapeDtypeStruct((1, NP), jnp.float32),
        ],
    )


def _k3_body(xn_hbm, src_hbm, dst_hbm, nin_hbm, agg_hbm, sp_hbm,
             sidx0, didx0, rows0, sidx1, didx1, rows1, tsidx, tdidx,
             didxs0, didxs1, ninv, sloc, agg_sh, sem0, sem1, semt,
             semi0, semi1, semsc0, semsc1, *, NP, EW, D):
    c = lax.axis_index("c")
    s = lax.axis_index("s")
    wid = c * _NS + s
    rpt = NP // _NS
    z16 = jnp.zeros((16,), jnp.float32)
    ncf = (EW // _G) & ~1
    tail = EW - ncf * _G
    ebase = wid * EW

    def zero_body(j, _):
        for u in range(8):
            sloc[pl.ds(j * 128 + u * 16, 16)] = z16
        return 0

    lax.fori_loop(0, NP // 128, zero_body, 0)

    def zrow_body(i, _):
        for u in range(D // 16):
            rows0[i, pl.ds(u * 16, 16)] = z16
        return 0

    lax.fori_loop(0, _G, zrow_body, 0)
    r0 = pl.multiple_of(s * rpt, 8)
    nfull, rem = rpt // _G, rpt % _G
    for t in range(nfull):
        pltpu.async_copy(rows0, agg_sh.at[pl.ds(r0 + t * _G, _G)], semt)
    if rem:
        pltpu.async_copy(rows0.at[pl.ds(0, rem)],
                         agg_sh.at[pl.ds(r0 + nfull * _G, rem)], semt)
    pltpu.sync_copy(nin_hbm, ninv)
    for t in range(nfull):
        pltpu.make_async_copy(rows0, agg_sh.at[pl.ds(r0, _G)], semt).wait()
    if rem:
        pltpu.make_async_copy(rows0.at[pl.ds(0, rem)],
                              agg_sh.at[pl.ds(r0, rem)], semt).wait()

    def fire_idx(k, sv, dv, semi):
        e0 = pl.multiple_of(ebase + k * _G, 8)
        pltpu.async_copy(src_hbm.at[pl.ds(e0, _G)], sv, semi)
        pltpu.async_copy(dst_hbm.at[pl.ds(e0, _G)], dv, semi)

    def wait_idx(sv, dv, semi):
        dummy = pl.multiple_of(ebase, 8)
        pltpu.make_async_copy(src_hbm.at[pl.ds(dummy, _G)], sv, semi).wait()
        pltpu.make_async_copy(dst_hbm.at[pl.ds(dummy, _G)], dv, semi).wait()

    def spass(sv, dv, n):
        for j in range(n // 16):
            si = sv[pl.ds(j * 16, 16)]
            di = dv[pl.ds(j * 16, 16)]
            vals = plsc.load_gather(ninv, [di])
            plsc.addupdate_scatter(sloc, [si], vals)

    fire_idx(0, sidx0, didx0, semi0)
    wait_idx(sidx0, didx0, semi0)
    pltpu.async_copy(xn_hbm.at[sidx0], rows0, sem0)
    fire_idx(1, sidx1, didx1, semi1)
    plsc.subcore_barrier()

    def phase(k, cur, nxt):
        (csi, cdi, crows, cdis, csemi, csemg, csemsc) = cur
        (nsi, ndi, nrows, ndis, nsemi, nsemg, nsemsc) = nxt

        @pl.when(k + 1 < ncf)
        def _():
            @pl.when(k >= 1)
            def _():
                pltpu.make_async_copy(nrows, agg_sh.at[ndis], nsemsc).wait()

            wait_idx(nsi, ndi, nsemi)
            pltpu.async_copy(xn_hbm.at[nsi], nrows, nsemg)

        spass(csi, cdi, _G)
        pltpu.make_async_copy(xn_hbm.at[csi], crows, csemg).wait()
        for j in range(_G // 16):
            cdis[pl.ds(j * 16, 16)] = cdi[pl.ds(j * 16, 16)]
        pltpu.async_copy(crows, agg_sh.at[cdis], csemsc, add=True)

        @pl.when(k + 2 < ncf)
        def _():
            fire_idx(k + 2, csi, cdi, csemi)

    bufA = (sidx0, didx0, rows0, didxs0, semi0, sem0, semsc0)
    bufB = (sidx1, didx1, rows1, didxs1, semi1, sem1, semsc1)

    def pair(p, _):
        phase(p * 2, bufA, bufB)
        phase(p * 2 + 1, bufB, bufA)
        return 0

    lax.fori_loop(0, ncf // 2, pair, 0)
    pltpu.make_async_copy(rows0, agg_sh.at[didxs0], semsc0).wait()
    pltpu.make_async_copy(rows1, agg_sh.at[didxs1], semsc1).wait()

    if tail:
        e0 = pl.multiple_of(ebase + ncf * _G, 8)
        pltpu.sync_copy(src_hbm.at[pl.ds(e0, tail)], tsidx)
        pltpu.sync_copy(dst_hbm.at[pl.ds(e0, tail)], tdidx)
        cp = pltpu.async_copy(xn_hbm.at[tsidx], rows0.at[pl.ds(0, tail)], semt)
        spass(tsidx, tdidx, tail)
        cp.wait()
        pltpu.sync_copy(rows0.at[pl.ds(0, tail)], agg_sh.at[tdidx], add=True)

    plsc.subcore_barrier()
    pltpu.sync_copy(agg_sh.at[pl.ds(r0, rpt)], agg_hbm.at[c, pl.ds(r0, rpt)])
    pltpu.sync_copy(sloc, sp_hbm.at[wid])


def _make_k3(NP, EW, D):
    tail = EW - ((EW // _G) & ~1) * _G
    mesh = plsc.VectorSubcoreMesh(core_axis_name="c", subcore_axis_name="s")
    return pl.kernel(
        functools.partial(_k3_body, NP=NP, EW=EW, D=D),
        out_type=(
            jax.ShapeDtypeStruct((_NC, NP, D), jnp.float32),
            jax.ShapeDtypeStruct((_NW, NP), jnp.float32),
        ),
        mesh=mesh,
        compiler_params=pltpu.CompilerParams(needs_layout_passes=False),
        scratch_types=[
            pltpu.VMEM((_G,), jnp.int32),
            pltpu.VMEM((_G,), jnp.int32),
            pltpu.VMEM((_G, D), jnp.float32),
            pltpu.VMEM((_G,), jnp.int32),
            pltpu.VMEM((_G,), jnp.int32),
            pltpu.VMEM((_G, D), jnp.float32),
            pltpu.VMEM((max(tail, 8),), jnp.int32),
            pltpu.VMEM((max(tail, 8),), jnp.int32),
            pltpu.VMEM((_G,), jnp.int32),
            pltpu.VMEM((_G,), jnp.int32),
            pltpu.VMEM((NP,), jnp.float32),
            pltpu.VMEM((NP,), jnp.float32),
            pltpu.VMEM_SHARED((NP, D), jnp.float32),
            pltpu.SemaphoreType.DMA,
            pltpu.SemaphoreType.DMA,
            pltpu.SemaphoreType.DMA,
            pltpu.SemaphoreType.DMA,
            pltpu.SemaphoreType.DMA,
            pltpu.SemaphoreType.DMA,
            pltpu.SemaphoreType.DMA,
        ],
    )


def _k4_body(agg_ref, sp_ref, no_ref, ni_ref, w1_ref, b1_ref,
             w2_ref, b2_ref, out_ref, vacc, *, nb, n_nodes):
    i = pl.program_id(0)

    @pl.when(i == 0)
    def _():
        vacc[...] = jnp.zeros_like(vacc)

    agg = agg_ref[0] + agg_ref[1]
    z = jnp.dot(agg, w1_ref[...], preferred_element_type=jnp.float32)
    h1 = jnp.maximum(z * ni_ref[0, :][:, None] + b1_ref[...], 0.0)
    w = no_ref[...] * jnp.sum(sp_ref[...], axis=0)[None, :]
    vacc[...] += jnp.dot(w, h1, preferred_element_type=jnp.float32)

    @pl.when(i == nb - 1)
    def _():
        v = vacc[...] * (1.0 / n_nodes)
        out_ref[...] = (
            jnp.dot(v, w2_ref[...], preferred_element_type=jnp.float32)
            + b2_ref[...]
        )


def _make_k4(NP, D, H, n_nodes):
    nb = NP // _BR
    return pl.pallas_call(
        functools.partial(_k4_body, nb=nb, n_nodes=n_nodes),
        grid=(nb,),
        in_specs=[
            pl.BlockSpec((_NC, _BR, D), lambda i: (0, i, 0)),
            pl.BlockSpec((_NW, _BR), lambda i: (0, i)),
            pl.BlockSpec((1, _BR), lambda i: (0, i)),
            pl.BlockSpec((1, _BR), lambda i: (0, i)),
            pl.BlockSpec((D, H), lambda i: (0, 0)),
            pl.BlockSpec((1, H), lambda i: (0, 0)),
            pl.BlockSpec((H, 128), lambda i: (0, 0)),
            pl.BlockSpec((1, 128), lambda i: (0, 0)),
        ],
        out_specs=pl.BlockSpec((1, 128), lambda i: (0, 0)),
        out_shape=jax.ShapeDtypeStruct((1, 128), jnp.float32),
        scratch_shapes=[pltpu.VMEM((1, 128), jnp.float32)],
    )


def kernel(x, edge_index, W1, b1, W2, b2):
    N, D = x.shape
    E = edge_index.shape[1]
    H = W1.shape[1]
    C = W2.shape[1]
    NP = -(-N // 2048) * 2048
    EW = E // _NW
    assert E % (_NW * 8) == 0 and NP % (_NS * 8) == 0 and NP % _BR == 0

    xpad = jnp.pad(x, ((0, NP - N), (0, 0)))
    src = edge_index[0]
    dst = edge_index[1]
    dpo, dpi = _make_k1(NP, EW)(src, dst)
    xn, no_, ni_ = _make_k2(NP, D)(dpo, dpi, xpad)
    agg2, spart = _make_k3(NP, EW, D)(xn, src, dst, ni_.reshape(NP))
    w2p = jnp.pad(W2, ((0, 0), (0, 128 - C)))
    b2p = jnp.pad(b2, (0, 128 - C)).reshape(1, 128)
    out = _make_k4(NP, D, H, N)(
        agg2, spart, no_, ni_, W1, b1.reshape(1, H), w2p, b2p)
    return out[:, :C]


# speedup vs baseline: 21.4099x; 1.0359x over previous
import functools

import jax
import jax.numpy as jnp
from jax import lax
from jax.experimental import pallas as pl
from jax.experimental.pallas import tpu as pltpu
from jax.experimental.pallas import tpu_sc as plsc

_NC = 2
_NS = 16
_NW = _NC * _NS
_G = 64
_BR = 1024


def _k1_body(src_hbm, dst_hbm, dpo_hbm, dpi_hbm, sv, dv, ho, hi, sems, semd,
             *, NP, EW):
    c = lax.axis_index("c")
    s = lax.axis_index("s")
    wid = c * _NS + s
    z16 = jnp.zeros((16,), jnp.float32)

    e0 = pl.multiple_of(wid * EW, 8)
    cps = pltpu.async_copy(src_hbm.at[pl.ds(e0, EW)], sv, sems)
    cpd = pltpu.async_copy(dst_hbm.at[pl.ds(e0, EW)], dv, semd)

    def zero_body(j, _):
        for u in range(8):
            ho[pl.ds(j * 128 + u * 16, 16)] = z16
            hi[pl.ds(j * 128 + u * 16, 16)] = z16
        return 0

    lax.fori_loop(0, NP // 128, zero_body, 0)
    cps.wait()
    cpd.wait()
    ones = jnp.ones((16,), jnp.float32)

    def edge_body(j, _):
        for u in range(5):
            si = sv[pl.ds(j * 80 + u * 16, 16)]
            di = dv[pl.ds(j * 80 + u * 16, 16)]
            plsc.addupdate_scatter(ho, [si], ones)
            plsc.addupdate_scatter(hi, [di], ones)
        return 0

    lax.fori_loop(0, EW // 80, edge_body, 0)
    pltpu.sync_copy(ho, dpo_hbm.at[wid])
    pltpu.sync_copy(hi, dpi_hbm.at[wid])


def _make_k1(NP, EW):
    mesh = plsc.VectorSubcoreMesh(core_axis_name="c", subcore_axis_name="s")
    return pl.kernel(
        functools.partial(_k1_body, NP=NP, EW=EW),
        out_type=(
            jax.ShapeDtypeStruct((_NW, NP), jnp.float32),
            jax.ShapeDtypeStruct((_NW, NP), jnp.float32),
        ),
        mesh=mesh,
        compiler_params=pltpu.CompilerParams(needs_layout_passes=False),
        scratch_types=[
            pltpu.VMEM((EW,), jnp.int32),
            pltpu.VMEM((EW,), jnp.int32),
            pltpu.VMEM((NP,), jnp.float32),
            pltpu.VMEM((NP,), jnp.float32),
            pltpu.SemaphoreType.DMA,
            pltpu.SemaphoreType.DMA,
        ],
    )


def _k2_body(dpo_ref, dpi_ref, x_ref, xn_ref, no_ref, ni_ref):
    deg_o = jnp.sum(dpo_ref[...], axis=0)
    deg_i = jnp.sum(dpi_ref[...], axis=0)
    no = lax.rsqrt(jnp.where(deg_o > 0.0, deg_o, 1.0))
    ni = lax.rsqrt(jnp.where(deg_i > 0.0, deg_i, 1.0))
    no_ref[...] = no[None, :]
    ni_ref[...] = ni[None, :]
    xn_ref[...] = x_ref[...] * no[:, None]


def _make_k2(NP, D):
    nb = NP // _BR
    return pl.pallas_call(
        _k2_body,
        grid=(nb,),
        in_specs=[
            pl.BlockSpec((_NW, _BR), lambda i: (0, i)),
            pl.BlockSpec((_NW, _BR), lambda i: (0, i)),
            pl.BlockSpec((_BR, D), lambda i: (i, 0)),
        ],
        out_specs=[
            pl.BlockSpec((_BR, D), lambda i: (i, 0)),
            pl.BlockSpec((1, _BR), lambda i: (0, i)),
            pl.BlockSpec((1, _BR), lambda i: (0, i)),
        ],
        out_shape=[
            jax.ShapeDtypeStruct((NP, D), jnp.float32),
            jax.ShapeDtypeStruct((1, NP), jnp.float32),
            jax.ShapeDtypeStruct((1, NP), jnp.float32),
        ],
    )


def _k3_body(xn_hbm, src_hbm, dst_hbm, nin_hbm, agg_hbm, sp_hbm,
             sidx0, didx0, rows0, sidx1, didx1, rows1, sidx2, didx2, rows2,
             tsidx, tdidx, didxs0, didxs1, didxs2, ninv, sloc, agg_sh,
             sem0, sem1, sem2, semt, semi0, semi1, semi2,
             semsc0, semsc1, semsc2, *, NP, EW, D):
    c = lax.axis_index("c")
    s = lax.axis_index("s")
    wid = c * _NS + s
    rpt = NP // _NS
    z16 = jnp.zeros((16,), jnp.float32)
    ncf = ((EW // _G) // 3) * 3
    tail = EW - ncf * _G
    ebase = wid * EW

    def zero_body(j, _):
        for u in range(8):
            sloc[pl.ds(j * 128 + u * 16, 16)] = z16
        return 0

    lax.fori_loop(0, NP // 128, zero_body, 0)

    def zrow_body(i, _):
        for u in range(D // 16):
            rows0[i, pl.ds(u * 16, 16)] = z16
        return 0

    lax.fori_loop(0, _G, zrow_body, 0)
    r0 = pl.multiple_of(s * rpt, 8)
    nfull, rem = rpt // _G, rpt % _G
    for t in range(nfull):
        pltpu.async_copy(rows0, agg_sh.at[pl.ds(r0 + t * _G, _G)], semt)
    if rem:
        pltpu.async_copy(rows0.at[pl.ds(0, rem)],
                         agg_sh.at[pl.ds(r0 + nfull * _G, rem)], semt)
    pltpu.sync_copy(nin_hbm, ninv)
    for t in range(nfull):
        pltpu.make_async_copy(rows0, agg_sh.at[pl.ds(r0, _G)], semt).wait()
    if rem:
        pltpu.make_async_copy(rows0.at[pl.ds(0, rem)],
                              agg_sh.at[pl.ds(r0, rem)], semt).wait()

    def fire_idx(k, sv, dv, semi):
        e0 = pl.multiple_of(ebase + k * _G, 8)
        pltpu.async_copy(src_hbm.at[pl.ds(e0, _G)], sv, semi)
        pltpu.async_copy(dst_hbm.at[pl.ds(e0, _G)], dv, semi)

    def wait_idx(sv, dv, semi):
        dummy = pl.multiple_of(ebase, 8)
        pltpu.make_async_copy(src_hbm.at[pl.ds(dummy, _G)], sv, semi).wait()
        pltpu.make_async_copy(dst_hbm.at[pl.ds(dummy, _G)], dv, semi).wait()

    def spass(sv, dv, n):
        for j in range(n // 16):
            si = sv[pl.ds(j * 16, 16)]
            di = dv[pl.ds(j * 16, 16)]
            vals = plsc.load_gather(ninv, [di])
            plsc.addupdate_scatter(sloc, [si], vals)

    fire_idx(0, sidx0, didx0, semi0)
    wait_idx(sidx0, didx0, semi0)
    pltpu.async_copy(xn_hbm.at[sidx0], rows0, sem0)
    fire_idx(1, sidx1, didx1, semi1)
    wait_idx(sidx1, didx1, semi1)
    pltpu.async_copy(xn_hbm.at[sidx1], rows1, sem1)
    fire_idx(2, sidx2, didx2, semi2)
    plsc.subcore_barrier()

    def phase(k, cur, nxt, nn):
        (csi, cdi, crows, cdis, csemi, csemg, csemsc) = cur
        (msi, mdi, mrows, mdis, msemi, msemg, msemsc) = nn

        @pl.when(k + 2 < ncf)
        def _():
            @pl.when(k >= 1)
            def _():
                pltpu.make_async_copy(mrows, agg_sh.at[mdis], msemsc).wait()

            wait_idx(msi, mdi, msemi)
            pltpu.async_copy(xn_hbm.at[msi], mrows, msemg)

        spass(csi, cdi, _G)
        pltpu.make_async_copy(xn_hbm.at[csi], crows, csemg).wait()
        for j in range(_G // 16):
            cdis[pl.ds(j * 16, 16)] = cdi[pl.ds(j * 16, 16)]
        pltpu.async_copy(crows, agg_sh.at[cdis], csemsc, add=True)

        @pl.when(k + 3 < ncf)
        def _():
            fire_idx(k + 3, csi, cdi, csemi)

    bufA = (sidx0, didx0, rows0, didxs0, semi0, sem0, semsc0)
    bufB = (sidx1, didx1, rows1, didxs1, semi1, sem1, semsc1)
    bufC = (sidx2, didx2, rows2, didxs2, semi2, sem2, semsc2)

    def triple(p, _):
        phase(p * 3, bufA, bufB, bufC)
        phase(p * 3 + 1, bufB, bufC, bufA)
        phase(p * 3 + 2, bufC, bufA, bufB)
        return 0

    lax.fori_loop(0, ncf // 3, triple, 0)
    pltpu.make_async_copy(rows0, agg_sh.at[didxs0], semsc0).wait()
    pltpu.make_async_copy(rows1, agg_sh.at[didxs1], semsc1).wait()
    pltpu.make_async_copy(rows2, agg_sh.at[didxs2], semsc2).wait()

    if tail:
        e0 = pl.multiple_of(ebase + ncf * _G, 8)
        pltpu.sync_copy(src_hbm.at[pl.ds(e0, tail)], tsidx)
        pltpu.sync_copy(dst_hbm.at[pl.ds(e0, tail)], tdidx)
        cp = pltpu.async_copy(xn_hbm.at[tsidx], rows0.at[pl.ds(0, tail)], semt)
        spass(tsidx, tdidx, tail)
        cp.wait()
        pltpu.sync_copy(rows0.at[pl.ds(0, tail)], agg_sh.at[tdidx], add=True)

    plsc.subcore_barrier()
    pltpu.sync_copy(agg_sh.at[pl.ds(r0, rpt)], agg_hbm.at[c, pl.ds(r0, rpt)])
    pltpu.sync_copy(sloc, sp_hbm.at[wid])


def _make_k3(NP, EW, D):
    tail = EW - ((EW // _G) // 3) * 3 * _G
    mesh = plsc.VectorSubcoreMesh(core_axis_name="c", subcore_axis_name="s")
    return pl.kernel(
        functools.partial(_k3_body, NP=NP, EW=EW, D=D),
        out_type=(
            jax.ShapeDtypeStruct((_NC, NP, D), jnp.float32),
            jax.ShapeDtypeStruct((_NW, NP), jnp.float32),
        ),
        mesh=mesh,
        compiler_params=pltpu.CompilerParams(needs_layout_passes=False),
        scratch_types=(
            [pltpu.VMEM((_G,), jnp.int32), pltpu.VMEM((_G,), jnp.int32),
             pltpu.VMEM((_G, D), jnp.float32)] * 3
            + [pltpu.VMEM((max(tail, 8),), jnp.int32),
               pltpu.VMEM((max(tail, 8),), jnp.int32)]
            + [pltpu.VMEM((_G,), jnp.int32)] * 3
            + [pltpu.VMEM((NP,), jnp.float32),
               pltpu.VMEM((NP,), jnp.float32),
               pltpu.VMEM_SHARED((NP, D), jnp.float32)]
            + [pltpu.SemaphoreType.DMA] * 10
        ),
    )


def _k4_body(agg_ref, sp_ref, no_ref, ni_ref, w1_ref, b1_ref,
             w2_ref, b2_ref, out_ref, vacc, *, nb, n_nodes):
    i = pl.program_id(0)

    @pl.when(i == 0)
    def _():
        vacc[...] = jnp.zeros_like(vacc)

    agg = agg_ref[0] + agg_ref[1]
    z = jnp.dot(agg, w1_ref[...], preferred_element_type=jnp.float32)
    h1 = jnp.maximum(z * ni_ref[0, :][:, None] + b1_ref[...], 0.0)
    w = no_ref[...] * jnp.sum(sp_ref[...], axis=0)[None, :]
    vacc[...] += jnp.dot(w, h1, preferred_element_type=jnp.float32)

    @pl.when(i == nb - 1)
    def _():
        v = vacc[...] * (1.0 / n_nodes)
        out_ref[...] = (
            jnp.dot(v, w2_ref[...], preferred_element_type=jnp.float32)
            + b2_ref[...]
        )


def _make_k4(NP, D, H, n_nodes):
    nb = NP // _BR
    return pl.pallas_call(
        functools.partial(_k4_body, nb=nb, n_nodes=n_nodes),
        grid=(nb,),
        in_specs=[
            pl.BlockSpec((_NC, _BR, D), lambda i: (0, i, 0)),
            pl.BlockSpec((_NW, _BR), lambda i: (0, i)),
            pl.BlockSpec((1, _BR), lambda i: (0, i)),
            pl.BlockSpec((1, _BR), lambda i: (0, i)),
            pl.BlockSpec((D, H), lambda i: (0, 0)),
            pl.BlockSpec((1, H), lambda i: (0, 0)),
            pl.BlockSpec((H, 128), lambda i: (0, 0)),
            pl.BlockSpec((1, 128), lambda i: (0, 0)),
        ],
        out_specs=pl.BlockSpec((1, 128), lambda i: (0, 0)),
        out_shape=jax.ShapeDtypeStruct((1, 128), jnp.float32),
        scratch_shapes=[pltpu.VMEM((1, 128), jnp.float32)],
    )


def kernel(x, edge_index, W1, b1, W2, b2):
    N, D = x.shape
    E = edge_index.shape[1]
    H = W1.shape[1]
    C = W2.shape[1]
    NP = -(-N // 2048) * 2048
    EW = E // _NW
    assert E % (_NW * 8) == 0 and NP % (_NS * 8) == 0 and NP % _BR == 0

    xpad = jnp.pad(x, ((0, NP - N), (0, 0)))
    src = edge_index[0]
    dst = edge_index[1]
    dpo, dpi = _make_k1(NP, EW)(src, dst)
    xn, no_, ni_ = _make_k2(NP, D)(dpo, dpi, xpad)
    agg2, spart = _make_k3(NP, EW, D)(xn, src, dst, ni_.reshape(NP))
    w2p = jnp.pad(W2, ((0, 0), (0, 128 - C)))
    b2p = jnp.pad(b2, (0, 128 - C)).reshape(1, 128)
    out = _make_k4(NP, D, H, N)(
        agg2, spart, no_, ni_, W1, b1.reshape(1, H), w2p, b2p)
    return out[:, :C]


# speedup vs baseline: 21.6269x; 1.0101x over previous
import functools

import jax
import jax.numpy as jnp
from jax import lax
from jax.experimental import pallas as pl
from jax.experimental.pallas import tpu as pltpu
from jax.experimental.pallas import tpu_sc as plsc

_NC = 2
_NS = 16
_NW = _NC * _NS
_G = 64
_BR = 1024


def _k1_body(src_hbm, dst_hbm, dpo_hbm, dpi_hbm, sv, dv, ho, hi, sems, semd,
             *, NP, EW):
    c = lax.axis_index("c")
    s = lax.axis_index("s")
    wid = c * _NS + s
    z16 = jnp.zeros((16,), jnp.float32)

    e0 = pl.multiple_of(wid * EW, 8)
    cps = pltpu.async_copy(src_hbm.at[pl.ds(e0, EW)], sv, sems)
    cpd = pltpu.async_copy(dst_hbm.at[pl.ds(e0, EW)], dv, semd)

    def zero_body(j, _):
        for u in range(8):
            ho[pl.ds(j * 128 + u * 16, 16)] = z16
            hi[pl.ds(j * 128 + u * 16, 16)] = z16
        return 0

    lax.fori_loop(0, NP // 128, zero_body, 0)
    cps.wait()
    cpd.wait()
    ones = jnp.ones((16,), jnp.float32)

    def edge_body(j, _):
        for u in range(5):
            si = sv[pl.ds(j * 80 + u * 16, 16)]
            di = dv[pl.ds(j * 80 + u * 16, 16)]
            plsc.addupdate_scatter(ho, [si], ones)
            plsc.addupdate_scatter(hi, [di], ones)
        return 0

    lax.fori_loop(0, EW // 80, edge_body, 0)
    pltpu.sync_copy(ho, dpo_hbm.at[wid])
    pltpu.sync_copy(hi, dpi_hbm.at[wid])


def _make_k1(NP, EW):
    mesh = plsc.VectorSubcoreMesh(core_axis_name="c", subcore_axis_name="s")
    return pl.kernel(
        functools.partial(_k1_body, NP=NP, EW=EW),
        out_type=(
            jax.ShapeDtypeStruct((_NW, NP), jnp.float32),
            jax.ShapeDtypeStruct((_NW, NP), jnp.float32),
        ),
        mesh=mesh,
        compiler_params=pltpu.CompilerParams(needs_layout_passes=False),
        scratch_types=[
            pltpu.VMEM((EW,), jnp.int32),
            pltpu.VMEM((EW,), jnp.int32),
            pltpu.VMEM((NP,), jnp.float32),
            pltpu.VMEM((NP,), jnp.float32),
            pltpu.SemaphoreType.DMA,
            pltpu.SemaphoreType.DMA,
        ],
    )


def _k2_body(dpo_ref, dpi_ref, x_ref, xn_ref, no_ref, ni_ref):
    deg_o = jnp.sum(dpo_ref[...], axis=0)
    deg_i = jnp.sum(dpi_ref[...], axis=0)
    no = lax.rsqrt(jnp.where(deg_o > 0.0, deg_o, 1.0))
    ni = lax.rsqrt(jnp.where(deg_i > 0.0, deg_i, 1.0))
    no_ref[...] = no[None, :]
    ni_ref[...] = ni[None, :]
    xn_ref[...] = x_ref[...] * no[:, None]


def _make_k2(NP, D):
    nb = NP // _BR
    return pl.pallas_call(
        _k2_body,
        grid=(nb,),
        in_specs=[
            pl.BlockSpec((_NW, _BR), lambda i: (0, i)),
            pl.BlockSpec((_NW, _BR), lambda i: (0, i)),
            pl.BlockSpec((_BR, D), lambda i: (i, 0)),
        ],
        out_specs=[
            pl.BlockSpec((_BR, D), lambda i: (i, 0)),
            pl.BlockSpec((1, _BR), lambda i: (0, i)),
            pl.BlockSpec((1, _BR), lambda i: (0, i)),
        ],
        out_shape=[
            jax.ShapeDtypeStruct((NP, D), jnp.float32),
            jax.ShapeDtypeStruct((1, NP), jnp.float32),
            jax.ShapeDtypeStruct((1, NP), jnp.float32),
        ],
    )


def _k3_body(xn_hbm, src_hbm, dst_hbm, nin_hbm, agg_hbm, sp_hbm,
             sidx0, didx0, rows0, sidx1, didx1, rows1, sidx2, didx2, rows2,
             tsidx, tdidx, didxs0, didxs1, didxs2, ninv, sloc, agg_sh,
             sem0, sem1, sem2, semt, semi0, semi1, semi2,
             semsc0, semsc1, semsc2, *, NP, EW, D):
    c = lax.axis_index("c")
    s = lax.axis_index("s")
    wid = c * _NS + s
    rpt = NP // _NS
    z16 = jnp.zeros((16,), jnp.float32)
    ncf = ((EW // _G) // 3) * 3
    tail = EW - ncf * _G
    ebase = wid * EW

    def zero_body(j, _):
        for u in range(8):
            sloc[pl.ds(j * 128 + u * 16, 16)] = z16
        return 0

    lax.fori_loop(0, NP // 128, zero_body, 0)

    def zrow_body(i, _):
        for u in range(D // 16):
            rows0[i, pl.ds(u * 16, 16)] = z16
        return 0

    lax.fori_loop(0, _G, zrow_body, 0)
    r0 = pl.multiple_of(s * rpt, 8)
    nfull, rem = rpt // _G, rpt % _G
    for t in range(nfull):
        pltpu.async_copy(rows0, agg_sh.at[pl.ds(r0 + t * _G, _G)], semt)
    if rem:
        pltpu.async_copy(rows0.at[pl.ds(0, rem)],
                         agg_sh.at[pl.ds(r0 + nfull * _G, rem)], semt)
    pltpu.sync_copy(nin_hbm, ninv)
    for t in range(nfull):
        pltpu.make_async_copy(rows0, agg_sh.at[pl.ds(r0, _G)], semt).wait()
    if rem:
        pltpu.make_async_copy(rows0.at[pl.ds(0, rem)],
                              agg_sh.at[pl.ds(r0, rem)], semt).wait()

    def fire_idx(k, sv, dv, semi):
        e0 = pl.multiple_of(ebase + k * _G, 8)
        pltpu.async_copy(src_hbm.at[pl.ds(e0, _G)], sv, semi)
        pltpu.async_copy(dst_hbm.at[pl.ds(e0, _G)], dv, semi)

    def wait_idx(sv, dv, semi):
        dummy = pl.multiple_of(ebase, 8)
        pltpu.make_async_copy(src_hbm.at[pl.ds(dummy, _G)], sv, semi).wait()
        pltpu.make_async_copy(dst_hbm.at[pl.ds(dummy, _G)], dv, semi).wait()

    def spass(sv, dv, n):
        for j in range(n // 16):
            si = sv[pl.ds(j * 16, 16)]
            di = dv[pl.ds(j * 16, 16)]
            vals = plsc.load_gather(ninv, [di])
            plsc.addupdate_scatter(sloc, [si], vals)

    fire_idx(0, sidx0, didx0, semi0)
    wait_idx(sidx0, didx0, semi0)
    pltpu.async_copy(xn_hbm.at[sidx0], rows0, sem0)
    fire_idx(1, sidx1, didx1, semi1)
    wait_idx(sidx1, didx1, semi1)
    pltpu.async_copy(xn_hbm.at[sidx1], rows1, sem1)
    fire_idx(2, sidx2, didx2, semi2)
    plsc.subcore_barrier()

    def phase(k, cur, nxt, nn):
        (csi, cdi, crows, cdis, csemi, csemg, csemsc) = cur
        (msi, mdi, mrows, mdis, msemi, msemg, msemsc) = nn

        @pl.when(k + 2 < ncf)
        def _():
            @pl.when(k >= 1)
            def _():
                pltpu.make_async_copy(mrows, agg_sh.at[mdis], msemsc).wait()

            wait_idx(msi, mdi, msemi)
            pltpu.async_copy(xn_hbm.at[msi], mrows, msemg)

        spass(csi, cdi, _G)
        pltpu.make_async_copy(xn_hbm.at[csi], crows, csemg).wait()
        for j in range(_G // 16):
            cdis[pl.ds(j * 16, 16)] = cdi[pl.ds(j * 16, 16)]
        pltpu.async_copy(crows, agg_sh.at[cdis], csemsc, add=True)

        @pl.when(k + 3 < ncf)
        def _():
            fire_idx(k + 3, csi, cdi, csemi)

    bufA = (sidx0, didx0, rows0, didxs0, semi0, sem0, semsc0)
    bufB = (sidx1, didx1, rows1, didxs1, semi1, sem1, semsc1)
    bufC = (sidx2, didx2, rows2, didxs2, semi2, sem2, semsc2)

    def triple(p, _):
        phase(p * 3, bufA, bufB, bufC)
        phase(p * 3 + 1, bufB, bufC, bufA)
        phase(p * 3 + 2, bufC, bufA, bufB)
        return 0

    lax.fori_loop(0, ncf // 3, triple, 0)
    pltpu.make_async_copy(rows0, agg_sh.at[didxs0], semsc0).wait()
    pltpu.make_async_copy(rows1, agg_sh.at[didxs1], semsc1).wait()
    pltpu.make_async_copy(rows2, agg_sh.at[didxs2], semsc2).wait()

    if tail:
        e0 = pl.multiple_of(ebase + ncf * _G, 8)
        pltpu.sync_copy(src_hbm.at[pl.ds(e0, tail)], tsidx)
        pltpu.sync_copy(dst_hbm.at[pl.ds(e0, tail)], tdidx)
        cp = pltpu.async_copy(xn_hbm.at[tsidx], rows0.at[pl.ds(0, tail)], semt)
        spass(tsidx, tdidx, tail)
        cp.wait()
        pltpu.sync_copy(rows0.at[pl.ds(0, tail)], agg_sh.at[tdidx], add=True)

    plsc.subcore_barrier()
    pltpu.sync_copy(agg_sh.at[pl.ds(r0, rpt)], agg_hbm.at[c, pl.ds(r0, rpt)])
    pltpu.sync_copy(sloc, sp_hbm.at[wid])


def _make_k3(NP, EW, D):
    tail = EW - ((EW // _G) // 3) * 3 * _G
    mesh = plsc.VectorSubcoreMesh(core_axis_name="c", subcore_axis_name="s")
    return pl.kernel(
        functools.partial(_k3_body, NP=NP, EW=EW, D=D),
        out_type=(
            jax.ShapeDtypeStruct((_NC, NP, D), jnp.float32),
            jax.ShapeDtypeStruct((_NW, NP), jnp.float32),
        ),
        mesh=mesh,
        compiler_params=pltpu.CompilerParams(needs_layout_passes=False),
        scratch_types=(
            [pltpu.VMEM((_G,), jnp.int32), pltpu.VMEM((_G,), jnp.int32),
             pltpu.VMEM((_G, D), jnp.float32)] * 3
            + [pltpu.VMEM((max(tail, 8),), jnp.int32),
               pltpu.VMEM((max(tail, 8),), jnp.int32)]
            + [pltpu.VMEM((_G,), jnp.int32)] * 3
            + [pltpu.VMEM((NP,), jnp.float32),
               pltpu.VMEM((NP,), jnp.float32),
               pltpu.VMEM_SHARED((NP, D), jnp.float32)]
            + [pltpu.SemaphoreType.DMA] * 10
        ),
    )


def _k4_body(agg_ref, sp_ref, no_ref, ni_ref, w1_ref, b1_ref,
             w2_ref, b2_ref, out_ref, vacc, *, nb, n_nodes):
    i = pl.program_id(0)

    @pl.when(i == 0)
    def _():
        vacc[...] = jnp.zeros_like(vacc)

    agg = agg_ref[0] + agg_ref[1]
    z = jnp.dot(agg, w1_ref[...], preferred_element_type=jnp.float32)
    h1 = jnp.maximum(z * ni_ref[0, :][:, None] + b1_ref[...], 0.0)
    w = no_ref[...] * jnp.sum(sp_ref[...], axis=0)[None, :]
    vacc[...] += jnp.dot(w, h1, preferred_element_type=jnp.float32)

    @pl.when(i == nb - 1)
    def _():
        v = vacc[...] * (1.0 / n_nodes)
        out_ref[...] = (
            jnp.dot(v, w2_ref[...], preferred_element_type=jnp.float32)
            + b2_ref[...]
        )


def _make_k4(NP, D, H, n_nodes):
    nb = NP // _BR
    return pl.pallas_call(
        functools.partial(_k4_body, nb=nb, n_nodes=n_nodes),
        grid=(nb,),
        in_specs=[
            pl.BlockSpec((_NC, _BR, D), lambda i: (0, i, 0)),
            pl.BlockSpec((_NW, _BR), lambda i: (0, i)),
            pl.BlockSpec((1, _BR), lambda i: (0, i)),
            pl.BlockSpec((1, _BR), lambda i: (0, i)),
            pl.BlockSpec((D, H), lambda i: (0, 0)),
            pl.BlockSpec((1, H), lambda i: (0, 0)),
            pl.BlockSpec((H, 128), lambda i: (0, 0)),
            pl.BlockSpec((1, 128), lambda i: (0, 0)),
        ],
        out_specs=pl.BlockSpec((1, 128), lambda i: (0, 0)),
        out_shape=jax.ShapeDtypeStruct((1, 128), jnp.float32),
        scratch_shapes=[pltpu.VMEM((1, 128), jnp.float32)],
    )


def kernel(x, edge_index, W1, b1, W2, b2):
    N, D = x.shape
    E = edge_index.shape[1]
    H = W1.shape[1]
    C = W2.shape[1]
    NP = -(-N // 2048) * 2048
    EW = E // _NW
    assert E % (_NW * 8) == 0 and NP % (_NS * 8) == 0 and NP % _BR == 0

    src = edge_index[0]
    dst = edge_index[1]
    dpo, dpi = _make_k1(NP, EW)(src, dst)
    xn, no_, ni_ = _make_k2(NP, D)(dpo, dpi, x)
    agg2, spart = _make_k3(NP, EW, D)(xn, src, dst, ni_.reshape(NP))
    w2p = jnp.pad(W2, ((0, 0), (0, 128 - C)))
    b2p = jnp.pad(b2, (0, 128 - C)).reshape(1, 128)
    out = _make_k4(NP, D, H, N)(
        agg2, spart, no_, ni_, W1, b1.reshape(1, H), w2p, b2p)
    return out[:, :C]
